# Initial kernel scaffold; baseline (speedup 1.0000x reference)
import jax
import jax.numpy as jnp
from jax.experimental import pallas as pl


def kernel(node_features, edge_list, kernel, attention_kernel, bias):
    raise NotImplementedError("write your pallas kernel here")


# speedup vs baseline: 24.0464x; 24.0464x over previous
import functools

import jax
import jax.numpy as jnp
from jax import lax
from jax.experimental import pallas as pl
from jax.experimental.pallas import tpu as pltpu
from jax.experimental.pallas import tpu_sc as plsc

NC = 2
NS = 16
L = 16
NW = NC * NS

CH = 128
SB = 128
SHIFT = 7


def _mesh():
    return plsc.VectorSubcoreMesh(core_axis_name="c", subcore_axis_name="s")


def _lane():
    return lax.broadcasted_iota(jnp.int32, (L,), 0)


def _zero_ref(ref, nvec):
    def b(i, _):
        ref[pl.ds(i * L, L)] = jnp.zeros((L,), jnp.float32)
        return 0
    lax.fori_loop(0, nvec, b, 0)


def _k0_body(x_ref, w_ref, a2_ref, keys_ref, s_ref):
    k = jnp.dot(x_ref[...], w_ref[...], preferred_element_type=jnp.float32)
    keys_ref[...] = k
    s_ref[...] = jnp.dot(k, a2_ref[...], preferred_element_type=jnp.float32)


def _k5_body(keys_ref, wp_ref, b_ref, o_ref):
    w = jnp.sum(wp_ref[...], axis=0)
    o_ref[...] = jnp.maximum(keys_ref[...] * w[:, None] + b_ref[...], 0.0)


def _make_k1(N, E, EW, nfull, tail):
    EWP = EW + (CH - tail if tail else 0)

    @functools.partial(
        pl.kernel, mesh=_mesh(),
        compiler_params=pltpu.CompilerParams(needs_layout_passes=False),
        out_type=jax.ShapeDtypeStruct((NW * N,), jnp.float32),
        scratch_types=[pltpu.VMEM((EWP,), jnp.int32),
                       pltpu.VMEM((EWP,), jnp.int32),
                       pltpu.VMEM((CH,), jnp.float32),
                       pltpu.VMEM((CH,), jnp.float32),
                       pltpu.VMEM((N,), jnp.float32),
                       pltpu.VMEM_SHARED((N,), jnp.float32),
                       pltpu.VMEM_SHARED((N,), jnp.float32),
                       pltpu.SemaphoreType.DMA],
    )
    def k1(src_hbm, dst_hbm, s1_hbm, s2_hbm, part_hbm,
           srcb, dstb, g1, g2, ngh, st1, st2, sem):
        wid = lax.axis_index("s") * NC + lax.axis_index("c")
        ebase = wid * EW

        @pl.when(lax.axis_index("s") == 0)
        def _():
            pltpu.sync_copy(s1_hbm, st1)
            pltpu.sync_copy(s2_hbm, st2)
        pltpu.sync_copy(src_hbm.at[pl.ds(ebase, EW)], srcb.at[pl.ds(0, EW)])
        pltpu.sync_copy(dst_hbm.at[pl.ds(ebase, EW)], dstb.at[pl.ds(0, EW)])
        if EWP > EW:
            for j in range((EWP - EW) // L):
                srcb[pl.ds(EW + j * L, L)] = jnp.zeros((L,), jnp.int32)
                dstb[pl.ds(EW + j * L, L)] = jnp.zeros((L,), jnp.int32)
        _zero_ref(ngh, N // L)
        plsc.subcore_barrier()

        def chunk(off, n):
            pltpu.async_copy(st1.at[srcb.at[pl.ds(off, CH)]], g1, sem).wait()
            pltpu.async_copy(st2.at[dstb.at[pl.ds(off, CH)]], g2, sem).wait()
            for j in range(n // L):
                sv = srcb[pl.ds(off + j * L, L)]
                m = jnp.exp(g1[pl.ds(j * L, L)] + g2[pl.ds(j * L, L)])
                plsc.addupdate_scatter(ngh, [sv], m)

        def body(i, _):
            chunk(i * CH, CH)
            return 0
        lax.fori_loop(0, nfull, body, 0)
        if tail:
            chunk(nfull * CH, tail)
        pltpu.sync_copy(ngh, part_hbm.at[pl.ds(wid * N, N)])

    assert nfull > 0
    return k1


def _make_k2(N, stripe):
    nstripe = N // stripe

    @functools.partial(
        pl.kernel, mesh=_mesh(),
        compiler_params=pltpu.CompilerParams(needs_layout_passes=False),
        out_type=jax.ShapeDtypeStruct((N,), jnp.float32),
        scratch_types=[pltpu.VMEM((NW * stripe,), jnp.float32),
                       pltpu.VMEM((stripe,), jnp.float32)],
    )
    def k2(part_hbm, out_hbm, pbuf, obuf):
        wid = lax.axis_index("s") * NC + lax.axis_index("c")

        @pl.when(wid < nstripe)
        def _():
            col0 = wid * stripe
            for w in range(NW):
                pltpu.sync_copy(part_hbm.at[pl.ds(w * N + col0, stripe)],
                                pbuf.at[pl.ds(w * stripe, stripe)])
            for j in range(stripe // L):
                acc = jnp.zeros((L,), jnp.float32)
                for w in range(NW):
                    acc = acc + pbuf[pl.ds(w * stripe + j * L, L)]
                obuf[pl.ds(j * L, L)] = acc
            pltpu.sync_copy(obuf, out_hbm.at[pl.ds(col0, stripe)])

    return k2


def _make_k3(N, E, EW, nfull, tail, NBP, PWP, DUMW):
    EWP = EW + (CH - tail if tail else 0)
    nvec = EW // L

    @functools.partial(
        pl.kernel, mesh=_mesh(),
        compiler_params=pltpu.CompilerParams(needs_layout_passes=False),
        out_type=[jax.ShapeDtypeStruct((NW * PWP,), jnp.int32),
                  jax.ShapeDtypeStruct((NW * PWP,), jnp.float32),
                  jax.ShapeDtypeStruct((NW * NBP,), jnp.int32)],
        scratch_types=[pltpu.VMEM((EWP,), jnp.int32),
                       pltpu.VMEM((EWP,), jnp.int32),
                       pltpu.VMEM((EW,), jnp.float32),
                       pltpu.VMEM((CH,), jnp.float32),
                       pltpu.VMEM((CH,), jnp.float32),
                       pltpu.VMEM((NBP * L,), jnp.int32),
                       pltpu.VMEM((NBP * L,), jnp.int32),
                       pltpu.VMEM((NBP,), jnp.int32),
                       pltpu.VMEM((PWP,), jnp.int32),
                       pltpu.VMEM((PWP,), jnp.float32),
                       pltpu.VMEM_SHARED((N,), jnp.float32),
                       pltpu.VMEM_SHARED((N,), jnp.float32),
                       pltpu.VMEM_SHARED((N,), jnp.float32),
                       pltpu.SemaphoreType.DMA],
    )
    def k3(src_hbm, dst_hbm, s1_hbm, s2_hbm, neigh_hbm, enti_hbm, enta_hbm,
           cum_hbm, srcb, dstb, cb, nbuf, mbuf, hist, curs, cumb, stgi, stga,
           stn, st1, st2, sem):
        wid = lax.axis_index("s") * NC + lax.axis_index("c")
        ebase = wid * EW
        lane = _lane()

        @pl.when(lax.axis_index("s") == 0)
        def _():
            pltpu.sync_copy(neigh_hbm, stn)
            pltpu.sync_copy(s1_hbm, st1)
            pltpu.sync_copy(s2_hbm, st2)

        pltpu.sync_copy(src_hbm.at[pl.ds(ebase, EW)], srcb.at[pl.ds(0, EW)])
        pltpu.sync_copy(dst_hbm.at[pl.ds(ebase, EW)], dstb.at[pl.ds(0, EW)])
        if EWP > EW:
            for j in range((EWP - EW) // L):
                srcb[pl.ds(EW + j * L, L)] = jnp.zeros((L,), jnp.int32)
                dstb[pl.ds(EW + j * L, L)] = jnp.zeros((L,), jnp.int32)

        plsc.subcore_barrier()

        def cchunk(off, n):
            pltpu.async_copy(st1.at[srcb.at[pl.ds(off, CH)]], mbuf,
                             sem).wait()
            pltpu.async_copy(st2.at[dstb.at[pl.ds(off, CH)]], nbuf,
                             sem).wait()
            for j in range(n // L):
                mbuf[pl.ds(j * L, L)] = jnp.exp(mbuf[pl.ds(j * L, L)] +
                                                nbuf[pl.ds(j * L, L)])
            pltpu.async_copy(stn.at[srcb.at[pl.ds(off, CH)]], nbuf,
                             sem).wait()
            for j in range(n // L):
                nb = nbuf[pl.ds(j * L, L)]
                m = mbuf[pl.ds(j * L, L)]
                safe = jnp.where(nb != 0.0, nb, jnp.ones((L,), jnp.float32))
                cb[pl.ds(off + j * L, L)] = jnp.where(
                    nb != 0.0, m / safe, jnp.zeros((L,), jnp.float32))

        def cbody(i, _):
            cchunk(i * CH, CH)
            return 0
        lax.fori_loop(0, nfull, cbody, 0)
        if tail:
            cchunk(nfull * CH, tail)

        def zh(i, _):
            hist[pl.ds(i * L, L)] = jnp.zeros((L,), jnp.int32)
            return 0
        lax.fori_loop(0, NBP, zh, 0)
        ones = jnp.ones((L,), jnp.int32)

        def hbody(i, _):
            s = srcb[pl.ds(i * L, L)]
            d = dstb[pl.ds(i * L, L)]
            plsc.addupdate_scatter(hist, [((d >> SHIFT) * 2) * L + lane], ones)
            plsc.addupdate_scatter(hist, [((s >> SHIFT) * 2 + 1) * L + lane],
                                   ones)
            return 0
        lax.fori_loop(0, nvec, hbody, 0)

        def sbody(b, run):
            h = hist[pl.ds(b * L, L)]
            inc = plsc.cumsum(h)
            curs[pl.ds(b * L, L)] = run + (inc - h)
            tot = jnp.sum(h)
            return run + (((tot + CH - 1) >> SHIFT) << SHIFT)
        lax.fori_loop(0, NBP, sbody, 0)

        def ebody(v, _):
            idx = (lane + v * L) * L
            cumb[pl.ds(v * L, L)] = plsc.load_gather(curs, [idx])
            return 0
        lax.fori_loop(0, NBP // L, ebody, 0)

        DUMT = SB * N
        def fbody(v, _):
            pos = v * L
            dums = DUMT + ((pos & 1023) + lane)
            r = pos & 15
            pada = ((DUMW + r) << 14) | (DUMW + r)
            stgi[pl.ds(pos, L)] = dums
            stga[pl.ds(pos, L)] = plsc.bitcast(
                jnp.full((L,), pada, jnp.int32), jnp.float32)
            return 0
        lax.fori_loop(0, PWP // L, fbody, 0)

        def pbody(i, _):
            s = srcb[pl.ds(i * L, L)]
            d = dstb[pl.ds(i * L, L)]
            c = cb[pl.ds(i * L, L)]
            ixw = (((d >> SHIFT) * 2) * L) + lane
            slot = plsc.load_gather(curs, [ixw])
            plsc.store_scatter(curs, [ixw], slot + 1)
            plsc.store_scatter(stgi, [slot], (d & (SB - 1)) * N + s)
            plsc.store_scatter(stga, [slot], c)
            ixr = (((s >> SHIFT) * 2 + 1) * L) + lane
            slot2 = plsc.load_gather(curs, [ixr])
            plsc.store_scatter(curs, [ixr], slot2 + 1)
            plsc.store_scatter(stgi, [slot2], (s & (SB - 1)) * N + d)
            plsc.store_scatter(stga, [slot2],
                               plsc.bitcast((s << 14) | d, jnp.float32))
            return 0
        lax.fori_loop(0, nvec, pbody, 0)

        pltpu.sync_copy(stgi, enti_hbm.at[pl.ds(wid * PWP, PWP)])
        pltpu.sync_copy(stga, enta_hbm.at[pl.ds(wid * PWP, PWP)])
        pltpu.sync_copy(cumb, cum_hbm.at[pl.ds(wid * NBP, NBP)])

    return k3


def _make_k4(N, NB, NBP, PWP, TSIZE, ZSPAN):
    NP16 = N + L
    ROWS = PWP // CH

    @functools.partial(
        pl.kernel, mesh=_mesh(),
        compiler_params=pltpu.CompilerParams(needs_layout_passes=False),
        out_type=jax.ShapeDtypeStruct((NW * N,), jnp.float32),
        scratch_types=[pltpu.VMEM((NW * NBP,), jnp.int32),
                       pltpu.VMEM((4, 1, CH), jnp.int32),
                       pltpu.VMEM((4, 1, CH), jnp.float32),
                       pltpu.VMEM((CH,), jnp.float32),
                       pltpu.VMEM((CH,), jnp.float32),
                       pltpu.VMEM((NP16,), jnp.float32),
                       pltpu.VMEM((2048,), jnp.float32),
                       pltpu.VMEM_SHARED((TSIZE,), jnp.float32),
                       pltpu.SemaphoreType.DMA,
                       pltpu.SemaphoreType.DMA,
                       pltpu.SemaphoreType.DMA,
                       pltpu.SemaphoreType.DMA,
                       pltpu.SemaphoreType.DMA,
                       pltpu.SemaphoreType.DMA,
                       pltpu.SemaphoreType.DMA,
                       pltpu.SemaphoreType.DMA],
    )
    def k4(enti_hbm, enta_hbm, cum_hbm, wpart_hbm,
           cums, ibuf, abuf, g128, z128, wloc, zbuf, tmat,
           si0, si1, si2, si3, sa0, sa1, sa2, sa3):
        semi = [si0, si1, si2, si3]
        sema = [sa0, sa1, sa2, sa3]
        c = lax.axis_index("c")
        s = lax.axis_index("s")
        wid = s * NC + c
        pltpu.sync_copy(cum_hbm, cums)
        _zero_ref(zbuf, 2048 // L)
        _zero_ref(z128, CH // L)
        _zero_ref(wloc, NP16 // L)

        nz = ZSPAN // 2048
        zt = ZSPAN - nz * 2048

        def zb(i, _):
            pltpu.sync_copy(zbuf, tmat.at[pl.ds(s * ZSPAN + i * 2048, 2048)])
            return 0
        lax.fori_loop(0, nz, zb, 0)
        if zt:
            pltpu.sync_copy(zbuf.at[pl.ds(0, zt)],
                            tmat.at[pl.ds(s * ZSPAN + nz * 2048, zt)])
        plsc.subcore_barrier()

        def extract(k):
            v = plsc.load_gather(cums, [jnp.full((L,), k, jnp.int32)])
            return jnp.max(v)

        def ranges(bent):
            w0 = 2 * s
            w1 = 2 * s + 1
            lo0 = extract(w0 * NBP + bent)
            hi0 = extract(w0 * NBP + bent + 1)
            lo1 = extract(w1 * NBP + bent)
            hi1 = extract(w1 * NBP + bent + 1)
            return (w0 * ROWS + (lo0 >> SHIFT), (hi0 - lo0) >> SHIFT,
                    w1 * ROWS + (lo1 >> SHIFT), (hi1 - lo1) >> SHIFT)

        def merged(rng, with_aux, proc):
            hr0, n0, hr1, n1 = rng
            n = n0 + n1

            def hrow(j):
                return jnp.where(j < n0, hr0 + j, hr1 + (j - n0))

            def grp(g, _):
                base = g * 4
                for k in range(4):
                    @pl.when(base + k < n)
                    def _(k=k):
                        r = hrow(base + k)
                        pltpu.async_copy(enti_hbm.at[r], ibuf.at[k], semi[k])
                        if with_aux:
                            pltpu.async_copy(enta_hbm.at[r], abuf.at[k],
                                             sema[k])
                for k in range(4):
                    @pl.when(base + k < n)
                    def _(k=k):
                        pltpu.make_async_copy(enti_hbm.at[0], ibuf.at[k],
                                              semi[k]).wait()
                        if with_aux:
                            pltpu.make_async_copy(enta_hbm.at[0], abuf.at[k],
                                                  sema[k]).wait()
                        proc(k)
                return 0
            lax.fori_loop(0, (n + 3) >> 2, grp, 0)

        def proc_a(k):
            pltpu.sync_copy(abuf.at[k, 0], tmat.at[ibuf.at[k, 0]], add=True)

        def proc_b(k):
            pltpu.sync_copy(tmat.at[ibuf.at[k, 0]], g128)
            for j in range(CH // L):
                co = g128[pl.ds(j * L, L)]
                ai = plsc.bitcast(abuf[k, 0, pl.ds(j * L, L)], jnp.int32)
                plsc.addupdate_scatter(wloc, [ai >> 14], co)
                plsc.addupdate_scatter(wloc, [ai & 0x3FFF], co)

        def proc_c(k):
            pltpu.sync_copy(z128, tmat.at[ibuf.at[k, 0]])

        nq = (NB + 1) // 2 - c * (1 if NB % 2 else 0)

        def qloop(i, _):
            q = 2 * i + c
            r_w = ranges(2 * q)
            r_r = ranges(2 * q + 1)
            merged(r_w, True, proc_a)
            plsc.subcore_barrier()
            merged(r_r, True, proc_b)
            plsc.subcore_barrier()
            merged(r_w, False, proc_c)
            plsc.subcore_barrier()
            return 0
        lax.fori_loop(0, nq, qloop, 0)
        pltpu.sync_copy(wloc.at[pl.ds(0, N)], wpart_hbm.at[pl.ds(wid * N, N)])

    return k4


def kernel(node_features, edge_list, kernel, attention_kernel, bias):
    N, F = node_features.shape
    D = kernel.shape[1]
    E = edge_list.shape[0]

    EW = E // NW
    assert EW * NW == E and EW % L == 0
    nfull, tail = divmod(EW, CH)
    NB = (N + SB - 1) // SB
    NBP = ((2 * NB + L - 1) // L) * L
    PWP = 2 * EW + NBP * (CH - 1)
    PWP = ((PWP + CH - 1) // CH) * CH
    TSIZE = SB * N + 1024
    ZSPAN = TSIZE // NS
    assert ZSPAN * NS == TSIZE and ZSPAN % 8 == 0
    NPAD = ((N + 511) // 512) * 512

    src = edge_list[:, 0]
    dst = edge_list[:, 1]
    ak2 = jnp.pad(attention_kernel.reshape(2, D).T, ((0, 0), (0, D - 2)))
    xpad = jnp.pad(node_features, ((0, NPAD - N), (0, 0)))

    keys, s12 = pl.pallas_call(
        _k0_body,
        grid=(NPAD // 512,),
        in_specs=[pl.BlockSpec((512, F), lambda i: (i, 0)),
                  pl.BlockSpec((F, D), lambda i: (0, 0)),
                  pl.BlockSpec((D, D), lambda i: (0, 0))],
        out_specs=[pl.BlockSpec((512, D), lambda i: (i, 0)),
                   pl.BlockSpec((512, D), lambda i: (i, 0))],
        out_shape=[jax.ShapeDtypeStruct((NPAD, D), jnp.float32),
                   jax.ShapeDtypeStruct((NPAD, D), jnp.float32)],
    )(xpad, kernel, ak2)

    s1 = s12[:N, 0]
    s2 = s12[:N, 1]

    parts = _make_k1(N, E, EW, nfull, tail)(src, dst, s1, s2)
    neigh = _make_k2(N, 400)(parts)
    enti, enta, cum = _make_k3(N, E, EW, nfull, tail, NBP, PWP, N)(
        src, dst, s1, s2, neigh)
    ROWS = PWP // CH
    wparts = _make_k4(N, NB, NBP, PWP, TSIZE, ZSPAN)(
        enti.reshape(NW * ROWS, 1, CH), enta.reshape(NW * ROWS, 1, CH), cum)

    wpad = jnp.pad(wparts.reshape(NW, N), ((0, 0), (0, NPAD - N)))
    out = pl.pallas_call(
        _k5_body,
        grid=(NPAD // 512,),
        in_specs=[pl.BlockSpec((512, D), lambda i: (i, 0)),
                  pl.BlockSpec((NW, 512), lambda i: (0, i)),
                  pl.BlockSpec((1, D), lambda i: (0, 0))],
        out_specs=pl.BlockSpec((512, D), lambda i: (i, 0)),
        out_shape=jax.ShapeDtypeStruct((NPAD, D), jnp.float32),
    )(keys, wpad, bias.reshape(1, D))
    return out[:N]


# speedup vs baseline: 25.2881x; 1.0516x over previous
import functools

import jax
import jax.numpy as jnp
from jax import lax
from jax.experimental import pallas as pl
from jax.experimental.pallas import tpu as pltpu
from jax.experimental.pallas import tpu_sc as plsc

NC = 2
NS = 16
L = 16
NW = NC * NS

CH = 128
SB = 128
SHIFT = 7


def _mesh():
    return plsc.VectorSubcoreMesh(core_axis_name="c", subcore_axis_name="s")


def _lane():
    return lax.broadcasted_iota(jnp.int32, (L,), 0)


def _zero_ref(ref, nvec):
    def b(i, _):
        ref[pl.ds(i * L, L)] = jnp.zeros((L,), jnp.float32)
        return 0
    lax.fori_loop(0, nvec, b, 0)


def _k0_body(x_ref, w_ref, a2_ref, keys_ref, s_ref):
    k = jnp.dot(x_ref[...], w_ref[...], preferred_element_type=jnp.float32)
    keys_ref[...] = k
    s_ref[...] = jnp.dot(k, a2_ref[...], preferred_element_type=jnp.float32)


def _k5_body(keys_ref, wp_ref, b_ref, o_ref):
    w = jnp.sum(wp_ref[...], axis=0)
    o_ref[...] = jnp.maximum(keys_ref[...] * w[:, None] + b_ref[...], 0.0)


def _make_k1(N, E, EW, nfull, tail):
    EWP = EW + (CH - tail if tail else 0)

    @functools.partial(
        pl.kernel, mesh=_mesh(),
        compiler_params=pltpu.CompilerParams(needs_layout_passes=False),
        out_type=jax.ShapeDtypeStruct((NW * N,), jnp.float32),
        scratch_types=[pltpu.VMEM((EWP,), jnp.int32),
                       pltpu.VMEM((EWP,), jnp.int32),
                       pltpu.VMEM((CH,), jnp.float32),
                       pltpu.VMEM((CH,), jnp.float32),
                       pltpu.VMEM((N,), jnp.float32),
                       pltpu.VMEM_SHARED((N,), jnp.float32),
                       pltpu.VMEM_SHARED((N,), jnp.float32),
                       pltpu.SemaphoreType.DMA],
    )
    def k1(src_hbm, dst_hbm, s1_hbm, s2_hbm, part_hbm,
           srcb, dstb, g1, g2, ngh, st1, st2, sem):
        wid = lax.axis_index("s") * NC + lax.axis_index("c")
        ebase = wid * EW

        @pl.when(lax.axis_index("s") == 0)
        def _():
            pltpu.sync_copy(s1_hbm, st1)
            pltpu.sync_copy(s2_hbm, st2)
        pltpu.sync_copy(src_hbm.at[pl.ds(ebase, EW)], srcb.at[pl.ds(0, EW)])
        pltpu.sync_copy(dst_hbm.at[pl.ds(ebase, EW)], dstb.at[pl.ds(0, EW)])
        if EWP > EW:
            for j in range((EWP - EW) // L):
                srcb[pl.ds(EW + j * L, L)] = jnp.zeros((L,), jnp.int32)
                dstb[pl.ds(EW + j * L, L)] = jnp.zeros((L,), jnp.int32)
        _zero_ref(ngh, N // L)
        plsc.subcore_barrier()

        def chunk(off, n):
            pltpu.async_copy(st1.at[srcb.at[pl.ds(off, CH)]], g1, sem).wait()
            pltpu.async_copy(st2.at[dstb.at[pl.ds(off, CH)]], g2, sem).wait()
            for j in range(n // L):
                sv = srcb[pl.ds(off + j * L, L)]
                m = jnp.exp(g1[pl.ds(j * L, L)] + g2[pl.ds(j * L, L)])
                plsc.addupdate_scatter(ngh, [sv], m)

        def body(i, _):
            chunk(i * CH, CH)
            return 0
        lax.fori_loop(0, nfull, body, 0)
        if tail:
            chunk(nfull * CH, tail)
        pltpu.sync_copy(ngh, part_hbm.at[pl.ds(wid * N, N)])

    assert nfull > 0
    return k1


def _make_k2(N, stripe):
    nstripe = N // stripe

    @functools.partial(
        pl.kernel, mesh=_mesh(),
        compiler_params=pltpu.CompilerParams(needs_layout_passes=False),
        out_type=jax.ShapeDtypeStruct((N,), jnp.float32),
        scratch_types=[pltpu.VMEM((NW * stripe,), jnp.float32),
                       pltpu.VMEM((stripe,), jnp.float32)],
    )
    def k2(part_hbm, out_hbm, pbuf, obuf):
        wid = lax.axis_index("s") * NC + lax.axis_index("c")

        @pl.when(wid < nstripe)
        def _():
            col0 = wid * stripe
            for w in range(NW):
                pltpu.sync_copy(part_hbm.at[pl.ds(w * N + col0, stripe)],
                                pbuf.at[pl.ds(w * stripe, stripe)])
            for j in range(stripe // L):
                acc = jnp.zeros((L,), jnp.float32)
                for w in range(NW):
                    acc = acc + pbuf[pl.ds(w * stripe + j * L, L)]
                obuf[pl.ds(j * L, L)] = acc
            pltpu.sync_copy(obuf, out_hbm.at[pl.ds(col0, stripe)])

    return k2


def _make_k3(N, E, EW, nfull, tail, NBP, PWP, DUMW):
    EWP = EW + (CH - tail if tail else 0)
    nvec = EW // L

    @functools.partial(
        pl.kernel, mesh=_mesh(),
        compiler_params=pltpu.CompilerParams(needs_layout_passes=False),
        out_type=[jax.ShapeDtypeStruct((NW * PWP,), jnp.int32),
                  jax.ShapeDtypeStruct((NW * PWP,), jnp.float32),
                  jax.ShapeDtypeStruct((NW * NBP,), jnp.int32)],
        scratch_types=[pltpu.VMEM((EWP,), jnp.int32),
                       pltpu.VMEM((EWP,), jnp.int32),
                       pltpu.VMEM((EW,), jnp.float32),
                       pltpu.VMEM((CH,), jnp.float32),
                       pltpu.VMEM((CH,), jnp.float32),
                       pltpu.VMEM((NBP * L,), jnp.int32),
                       pltpu.VMEM((NBP * L,), jnp.int32),
                       pltpu.VMEM((NBP,), jnp.int32),
                       pltpu.VMEM((PWP,), jnp.int32),
                       pltpu.VMEM((PWP,), jnp.float32),
                       pltpu.VMEM((640,), jnp.float32),
                       pltpu.VMEM((640,), jnp.float32),
                       pltpu.VMEM_SHARED((N,), jnp.float32),
                       pltpu.VMEM_SHARED((N,), jnp.float32),
                       pltpu.VMEM_SHARED((N,), jnp.float32),
                       pltpu.SemaphoreType.DMA],
    )
    def k3(src_hbm, dst_hbm, s1_hbm, s2_hbm, part_hbm, enti_hbm, enta_hbm,
           cum_hbm, srcb, dstb, cb, nbuf, mbuf, hist, curs, cumb, stgi, stga,
           pb, ab, stn, st1, st2, sem):
        sid = lax.axis_index("s")
        wid = sid * NC + lax.axis_index("c")
        ebase = wid * EW
        lane = _lane()

        @pl.when(sid == 0)
        def _():
            pltpu.sync_copy(s1_hbm, st1)
            pltpu.sync_copy(s2_hbm, st2)

        def stripe(col0, ln):
            for j in range(ln // L):
                ab[pl.ds(j * L, L)] = jnp.zeros((L,), jnp.float32)
            for w in range(NW):
                pltpu.sync_copy(part_hbm.at[pl.ds(w * N + col0, ln)],
                                pb.at[pl.ds(0, ln)])
                for j in range(ln // L):
                    ab[pl.ds(j * L, L)] = (ab[pl.ds(j * L, L)] +
                                           pb[pl.ds(j * L, L)])
            pltpu.sync_copy(ab.at[pl.ds(0, ln)], stn.at[pl.ds(col0, ln)])

        @pl.when(sid < NS - 1)
        def _():
            stripe(sid * 640, 640)

        @pl.when(sid == NS - 1)
        def _():
            stripe((NS - 1) * 640, N - (NS - 1) * 640)

        pltpu.sync_copy(src_hbm.at[pl.ds(ebase, EW)], srcb.at[pl.ds(0, EW)])
        pltpu.sync_copy(dst_hbm.at[pl.ds(ebase, EW)], dstb.at[pl.ds(0, EW)])
        if EWP > EW:
            for j in range((EWP - EW) // L):
                srcb[pl.ds(EW + j * L, L)] = jnp.zeros((L,), jnp.int32)
                dstb[pl.ds(EW + j * L, L)] = jnp.zeros((L,), jnp.int32)

        plsc.subcore_barrier()

        def cchunk(off, n):
            pltpu.async_copy(st1.at[srcb.at[pl.ds(off, CH)]], mbuf,
                             sem).wait()
            pltpu.async_copy(st2.at[dstb.at[pl.ds(off, CH)]], nbuf,
                             sem).wait()
            for j in range(n // L):
                mbuf[pl.ds(j * L, L)] = jnp.exp(mbuf[pl.ds(j * L, L)] +
                                                nbuf[pl.ds(j * L, L)])
            pltpu.async_copy(stn.at[srcb.at[pl.ds(off, CH)]], nbuf,
                             sem).wait()
            for j in range(n // L):
                nb = nbuf[pl.ds(j * L, L)]
                m = mbuf[pl.ds(j * L, L)]
                safe = jnp.where(nb != 0.0, nb, jnp.ones((L,), jnp.float32))
                cb[pl.ds(off + j * L, L)] = jnp.where(
                    nb != 0.0, m / safe, jnp.zeros((L,), jnp.float32))

        def cbody(i, _):
            cchunk(i * CH, CH)
            return 0
        lax.fori_loop(0, nfull, cbody, 0)
        if tail:
            cchunk(nfull * CH, tail)

        def zh(i, _):
            hist[pl.ds(i * L, L)] = jnp.zeros((L,), jnp.int32)
            return 0
        lax.fori_loop(0, NBP, zh, 0)
        ones = jnp.ones((L,), jnp.int32)

        def hbody(i, _):
            s = srcb[pl.ds(i * L, L)]
            d = dstb[pl.ds(i * L, L)]
            plsc.addupdate_scatter(hist, [((d >> SHIFT) * 2) * L + lane], ones)
            plsc.addupdate_scatter(hist, [((s >> SHIFT) * 2 + 1) * L + lane],
                                   ones)
            return 0
        lax.fori_loop(0, nvec, hbody, 0)

        def sbody(b, run):
            h = hist[pl.ds(b * L, L)]
            inc = plsc.cumsum(h)
            curs[pl.ds(b * L, L)] = run + (inc - h)
            tot = jnp.sum(h)
            return run + (((tot + CH - 1) >> SHIFT) << SHIFT)
        lax.fori_loop(0, NBP, sbody, 0)

        def ebody(v, _):
            idx = (lane + v * L) * L
            cumb[pl.ds(v * L, L)] = plsc.load_gather(curs, [idx])
            return 0
        lax.fori_loop(0, NBP // L, ebody, 0)

        DUMT = SB * N
        def fbody(v, _):
            pos = v * L
            dums = DUMT + ((pos & 1023) + lane)
            r = pos & 15
            pada = ((DUMW + r) << 14) | (DUMW + r)
            stgi[pl.ds(pos, L)] = dums
            stga[pl.ds(pos, L)] = plsc.bitcast(
                jnp.full((L,), pada, jnp.int32), jnp.float32)
            return 0
        lax.fori_loop(0, PWP // L, fbody, 0)

        def pbody(i, _):
            s = srcb[pl.ds(i * L, L)]
            d = dstb[pl.ds(i * L, L)]
            c = cb[pl.ds(i * L, L)]
            ixw = (((d >> SHIFT) * 2) * L) + lane
            slot = plsc.load_gather(curs, [ixw])
            plsc.store_scatter(curs, [ixw], slot + 1)
            plsc.store_scatter(stgi, [slot], (d & (SB - 1)) * N + s)
            plsc.store_scatter(stga, [slot], c)
            ixr = (((s >> SHIFT) * 2 + 1) * L) + lane
            slot2 = plsc.load_gather(curs, [ixr])
            plsc.store_scatter(curs, [ixr], slot2 + 1)
            plsc.store_scatter(stgi, [slot2], (s & (SB - 1)) * N + d)
            plsc.store_scatter(stga, [slot2],
                               plsc.bitcast((s << 14) | d, jnp.float32))
            return 0
        lax.fori_loop(0, nvec, pbody, 0)

        pltpu.sync_copy(stgi, enti_hbm.at[pl.ds(wid * PWP, PWP)])
        pltpu.sync_copy(stga, enta_hbm.at[pl.ds(wid * PWP, PWP)])
        pltpu.sync_copy(cumb, cum_hbm.at[pl.ds(wid * NBP, NBP)])

    return k3


def _make_k4(N, NB, NBP, PWP, TSIZE, ZSPAN):
    NP16 = N + L
    ROWS = PWP // CH

    @functools.partial(
        pl.kernel, mesh=_mesh(),
        compiler_params=pltpu.CompilerParams(needs_layout_passes=False),
        out_type=jax.ShapeDtypeStruct((NW * N,), jnp.float32),
        scratch_types=[pltpu.VMEM((NW * NBP,), jnp.int32),
                       pltpu.VMEM((4, 1, CH), jnp.int32),
                       pltpu.VMEM((4, 1, CH), jnp.float32),
                       pltpu.VMEM((16, 1, CH), jnp.int32),
                       pltpu.VMEM((CH,), jnp.float32),
                       pltpu.VMEM((CH,), jnp.float32),
                       pltpu.VMEM((NP16,), jnp.float32),
                       pltpu.VMEM((2048,), jnp.float32),
                       pltpu.VMEM_SHARED((TSIZE,), jnp.float32),
                       pltpu.SemaphoreType.DMA,
                       pltpu.SemaphoreType.DMA,
                       pltpu.SemaphoreType.DMA,
                       pltpu.SemaphoreType.DMA,
                       pltpu.SemaphoreType.DMA,
                       pltpu.SemaphoreType.DMA,
                       pltpu.SemaphoreType.DMA,
                       pltpu.SemaphoreType.DMA],
    )
    def k4(enti_hbm, enta_hbm, cum_hbm, wpart_hbm,
           cums, ibuf, abuf, stash, g128, z128, wloc, zbuf, tmat,
           si0, si1, si2, si3, sa0, sa1, sa2, sa3):
        semi = [si0, si1, si2, si3]
        sema = [sa0, sa1, sa2, sa3]
        c = lax.axis_index("c")
        s = lax.axis_index("s")
        wid = s * NC + c
        pltpu.sync_copy(cum_hbm, cums)
        _zero_ref(zbuf, 2048 // L)
        _zero_ref(z128, CH // L)
        _zero_ref(wloc, NP16 // L)

        nz = ZSPAN // 2048
        zt = ZSPAN - nz * 2048

        def zb(i, _):
            pltpu.sync_copy(zbuf, tmat.at[pl.ds(s * ZSPAN + i * 2048, 2048)])
            return 0
        lax.fori_loop(0, nz, zb, 0)
        if zt:
            pltpu.sync_copy(zbuf.at[pl.ds(0, zt)],
                            tmat.at[pl.ds(s * ZSPAN + nz * 2048, zt)])
        plsc.subcore_barrier()

        def extract(k):
            v = plsc.load_gather(cums, [jnp.full((L,), k, jnp.int32)])
            return jnp.max(v)

        def ranges(bent):
            w0 = 2 * s
            w1 = 2 * s + 1
            lo0 = extract(w0 * NBP + bent)
            hi0 = extract(w0 * NBP + bent + 1)
            lo1 = extract(w1 * NBP + bent)
            hi1 = extract(w1 * NBP + bent + 1)
            return (w0 * ROWS + (lo0 >> SHIFT), (hi0 - lo0) >> SHIFT,
                    w1 * ROWS + (lo1 >> SHIFT), (hi1 - lo1) >> SHIFT)

        def hrow_of(rng):
            hr0, n0, hr1, n1 = rng

            def hrow(j):
                return jnp.where(j < n0, hr0 + j, hr1 + (j - n0))
            return hrow, n0 + n1

        def fire(rng, base, with_aux):
            hrow, n = hrow_of(rng)
            for k in range(4):
                @pl.when(base + k < n)
                def _(k=k):
                    r = hrow(base + k)
                    pltpu.async_copy(enti_hbm.at[r], ibuf.at[k], semi[k])
                    if with_aux:
                        pltpu.async_copy(enta_hbm.at[r], abuf.at[k], sema[k])

        def merged(rng, with_aux, proc, prefired=False):
            hrow, n = hrow_of(rng)

            def grp(g, _):
                base = g * 4
                if prefired:
                    @pl.when(g > 0)
                    def _():
                        fire(rng, base, with_aux)
                else:
                    fire(rng, base, with_aux)
                for k in range(4):
                    @pl.when(base + k < n)
                    def _(k=k):
                        pltpu.make_async_copy(enti_hbm.at[0], ibuf.at[k],
                                              semi[k]).wait()
                        if with_aux:
                            pltpu.make_async_copy(enta_hbm.at[0], abuf.at[k],
                                                  sema[k]).wait()
                        proc(k, base + k)
                return 0
            lax.fori_loop(0, (n + 3) >> 2, grp, 0)

        def proc_a(k, j):
            pltpu.sync_copy(abuf.at[k, 0], tmat.at[ibuf.at[k, 0]], add=True)

            @pl.when(j < 16)
            def _():
                for j2 in range(CH // L):
                    stash[j, 0, pl.ds(j2 * L, L)] = ibuf[k, 0,
                                                        pl.ds(j2 * L, L)]

        def proc_b(k, j):
            pltpu.sync_copy(tmat.at[ibuf.at[k, 0]], g128)
            for j2 in range(CH // L):
                co = g128[pl.ds(j2 * L, L)]
                ai = plsc.bitcast(abuf[k, 0, pl.ds(j2 * L, L)], jnp.int32)
                plsc.addupdate_scatter(wloc, [ai >> 14], co)
                plsc.addupdate_scatter(wloc, [ai & 0x3FFF], co)

        def phase_c(rng):
            hrow, n = hrow_of(rng)
            ns = jnp.minimum(n, 16)

            def cs(j, _):
                pltpu.sync_copy(z128, tmat.at[stash.at[j, 0]])
                return 0
            lax.fori_loop(0, ns, cs, 0)

            def ct(j, _):
                pltpu.sync_copy(enti_hbm.at[hrow(j)], ibuf.at[0])
                pltpu.sync_copy(z128, tmat.at[ibuf.at[0, 0]])
                return 0
            lax.fori_loop(ns, n, ct, 0)

        nq = (NB + 1) // 2 - c * (1 if NB % 2 else 0)

        def qloop(i, _):
            q = 2 * i + c
            r_w = ranges(2 * q)
            r_r = ranges(2 * q + 1)
            merged(r_w, True, proc_a)
            fire(r_r, 0, True)
            plsc.subcore_barrier()
            merged(r_r, True, proc_b, prefired=True)
            plsc.subcore_barrier()
            phase_c(r_w)
            plsc.subcore_barrier()
            return 0
        lax.fori_loop(0, nq, qloop, 0)
        pltpu.sync_copy(wloc.at[pl.ds(0, N)], wpart_hbm.at[pl.ds(wid * N, N)])

    return k4


def kernel(node_features, edge_list, kernel, attention_kernel, bias):
    N, F = node_features.shape
    D = kernel.shape[1]
    E = edge_list.shape[0]

    EW = E // NW
    assert EW * NW == E and EW % L == 0
    nfull, tail = divmod(EW, CH)
    NB = (N + SB - 1) // SB
    NBP = ((2 * NB + L - 1) // L) * L
    PWP = 2 * EW + NBP * (CH - 1)
    PWP = ((PWP + CH - 1) // CH) * CH
    TSIZE = SB * N + 1024
    ZSPAN = TSIZE // NS
    assert ZSPAN * NS == TSIZE and ZSPAN % 8 == 0
    NPAD = ((N + 511) // 512) * 512

    src = edge_list[:, 0]
    dst = edge_list[:, 1]
    ak2 = jnp.pad(attention_kernel.reshape(2, D).T, ((0, 0), (0, D - 2)))
    xpad = jnp.pad(node_features, ((0, NPAD - N), (0, 0)))

    keys, s12 = pl.pallas_call(
        _k0_body,
        grid=(NPAD // 512,),
        in_specs=[pl.BlockSpec((512, F), lambda i: (i, 0)),
                  pl.BlockSpec((F, D), lambda i: (0, 0)),
                  pl.BlockSpec((D, D), lambda i: (0, 0))],
        out_specs=[pl.BlockSpec((512, D), lambda i: (i, 0)),
                   pl.BlockSpec((512, D), lambda i: (i, 0))],
        out_shape=[jax.ShapeDtypeStruct((NPAD, D), jnp.float32),
                   jax.ShapeDtypeStruct((NPAD, D), jnp.float32)],
    )(xpad, kernel, ak2)

    s1 = s12[:N, 0]
    s2 = s12[:N, 1]

    parts = _make_k1(N, E, EW, nfull, tail)(src, dst, s1, s2)
    enti, enta, cum = _make_k3(N, E, EW, nfull, tail, NBP, PWP, N)(
        src, dst, s1, s2, parts)
    ROWS = PWP // CH
    wparts = _make_k4(N, NB, NBP, PWP, TSIZE, ZSPAN)(
        enti.reshape(NW * ROWS, 1, CH), enta.reshape(NW * ROWS, 1, CH), cum)

    wpad = jnp.pad(wparts.reshape(NW, N), ((0, 0), (0, NPAD - N)))
    out = pl.pallas_call(
        _k5_body,
        grid=(NPAD // 512,),
        in_specs=[pl.BlockSpec((512, D), lambda i: (i, 0)),
                  pl.BlockSpec((NW, 512), lambda i: (0, i)),
                  pl.BlockSpec((1, D), lambda i: (0, 0))],
        out_specs=pl.BlockSpec((512, D), lambda i: (i, 0)),
        out_shape=jax.ShapeDtypeStruct((NPAD, D), jnp.float32),
    )(keys, wpad, bias.reshape(1, D))
    return out[:N]


# speedup vs baseline: 27.6599x; 1.0938x over previous
import functools

import jax
import jax.numpy as jnp
from jax import lax
from jax.experimental import pallas as pl
from jax.experimental.pallas import tpu as pltpu
from jax.experimental.pallas import tpu_sc as plsc

NC = 2
NS = 16
L = 16
NW = NC * NS

CH = 128
SB = 128
SHIFT = 7


def _mesh():
    return plsc.VectorSubcoreMesh(core_axis_name="c", subcore_axis_name="s")


def _lane():
    return lax.broadcasted_iota(jnp.int32, (L,), 0)


def _zero_ref(ref, nvec):
    def b(i, _):
        ref[pl.ds(i * L, L)] = jnp.zeros((L,), jnp.float32)
        return 0
    lax.fori_loop(0, nvec, b, 0)


def _k0_body(x_ref, w_ref, a2_ref, keys_ref, s_ref):
    k = jnp.dot(x_ref[...], w_ref[...], preferred_element_type=jnp.float32)
    keys_ref[...] = k
    s_ref[...] = jnp.dot(k, a2_ref[...], preferred_element_type=jnp.float32)


def _k5_body(keys_ref, wp_ref, b_ref, o_ref):
    w = jnp.sum(wp_ref[...], axis=0)
    o_ref[...] = jnp.maximum(keys_ref[...] * w[:, None] + b_ref[...], 0.0)


def _make_k1(N, E, EW, nfull, tail):
    EWP = EW + (CH - tail if tail else 0)

    @functools.partial(
        pl.kernel, mesh=_mesh(),
        compiler_params=pltpu.CompilerParams(needs_layout_passes=False),
        out_type=jax.ShapeDtypeStruct((NW * N,), jnp.float32),
        scratch_types=[pltpu.VMEM((EWP,), jnp.int32),
                       pltpu.VMEM((EWP,), jnp.int32),
                       pltpu.VMEM((CH,), jnp.float32),
                       pltpu.VMEM((CH,), jnp.float32),
                       pltpu.VMEM((N,), jnp.float32),
                       pltpu.VMEM_SHARED((N,), jnp.float32),
                       pltpu.VMEM_SHARED((N,), jnp.float32),
                       pltpu.SemaphoreType.DMA,
                       pltpu.SemaphoreType.DMA],
    )
    def k1(src_hbm, dst_hbm, s1_hbm, s2_hbm, part_hbm,
           srcb, dstb, g1, g2, ngh, st1, st2, sem, sem2):
        wid = lax.axis_index("s") * NC + lax.axis_index("c")
        ebase = wid * EW

        @pl.when(lax.axis_index("s") == 0)
        def _():
            pltpu.sync_copy(s1_hbm, st1)
            pltpu.sync_copy(s2_hbm, st2)
        pltpu.sync_copy(src_hbm.at[pl.ds(ebase, EW)], srcb.at[pl.ds(0, EW)])
        pltpu.sync_copy(dst_hbm.at[pl.ds(ebase, EW)], dstb.at[pl.ds(0, EW)])
        if EWP > EW:
            for j in range((EWP - EW) // L):
                srcb[pl.ds(EW + j * L, L)] = jnp.zeros((L,), jnp.int32)
                dstb[pl.ds(EW + j * L, L)] = jnp.zeros((L,), jnp.int32)
        _zero_ref(ngh, N // L)
        plsc.subcore_barrier()

        def chunk(off, n):
            h1 = pltpu.async_copy(st1.at[srcb.at[pl.ds(off, CH)]], g1, sem)
            h2 = pltpu.async_copy(st2.at[dstb.at[pl.ds(off, CH)]], g2, sem2)
            h1.wait()
            h2.wait()
            for j in range(n // L):
                sv = srcb[pl.ds(off + j * L, L)]
                m = jnp.exp(g1[pl.ds(j * L, L)] + g2[pl.ds(j * L, L)])
                plsc.addupdate_scatter(ngh, [sv], m)

        def body(i, _):
            chunk(i * CH, CH)
            return 0
        lax.fori_loop(0, nfull, body, 0)
        if tail:
            chunk(nfull * CH, tail)
        pltpu.sync_copy(ngh, part_hbm.at[pl.ds(wid * N, N)])

    assert nfull > 0
    return k1


def _make_k3(N, E, EW, nfull, tail, NBP, PWP, DUMW):
    EWP = EW + (CH - tail if tail else 0)
    nvec = EW // L

    @functools.partial(
        pl.kernel, mesh=_mesh(),
        compiler_params=pltpu.CompilerParams(needs_layout_passes=False),
        out_type=[jax.ShapeDtypeStruct((NW * PWP,), jnp.int32),
                  jax.ShapeDtypeStruct((NW * PWP,), jnp.float32),
                  jax.ShapeDtypeStruct((NW * NBP,), jnp.int32)],
        scratch_types=[pltpu.VMEM((EWP,), jnp.int32),
                       pltpu.VMEM((EWP,), jnp.int32),
                       pltpu.VMEM((EW,), jnp.float32),
                       pltpu.VMEM((CH,), jnp.float32),
                       pltpu.VMEM((CH,), jnp.float32),
                       pltpu.VMEM((NBP * L,), jnp.int32),
                       pltpu.VMEM((NBP * L,), jnp.int32),
                       pltpu.VMEM((NBP,), jnp.int32),
                       pltpu.VMEM((PWP,), jnp.int32),
                       pltpu.VMEM((PWP,), jnp.float32),
                       pltpu.VMEM((640,), jnp.float32),
                       pltpu.VMEM((640,), jnp.float32),
                       pltpu.VMEM((CH,), jnp.float32),
                       pltpu.VMEM_SHARED((N,), jnp.float32),
                       pltpu.VMEM_SHARED((N,), jnp.float32),
                       pltpu.VMEM_SHARED((N,), jnp.float32),
                       pltpu.SemaphoreType.DMA,
                       pltpu.SemaphoreType.DMA,
                       pltpu.SemaphoreType.DMA],
    )
    def k3(src_hbm, dst_hbm, s1_hbm, s2_hbm, part_hbm, enti_hbm, enta_hbm,
           cum_hbm, srcb, dstb, cb, nbuf, mbuf, hist, curs, cumb, stgi, stga,
           pb, ab, gn2, stn, st1, st2, sem, sem2, sem3):
        sid = lax.axis_index("s")
        wid = sid * NC + lax.axis_index("c")
        ebase = wid * EW
        lane = _lane()

        @pl.when(sid == 0)
        def _():
            pltpu.sync_copy(s1_hbm, st1)
            pltpu.sync_copy(s2_hbm, st2)

        def stripe(col0, ln):
            for j in range(ln // L):
                ab[pl.ds(j * L, L)] = jnp.zeros((L,), jnp.float32)
            for w in range(NW):
                pltpu.sync_copy(part_hbm.at[pl.ds(w * N + col0, ln)],
                                pb.at[pl.ds(0, ln)])
                for j in range(ln // L):
                    ab[pl.ds(j * L, L)] = (ab[pl.ds(j * L, L)] +
                                           pb[pl.ds(j * L, L)])
            pltpu.sync_copy(ab.at[pl.ds(0, ln)], stn.at[pl.ds(col0, ln)])

        @pl.when(sid < NS - 1)
        def _():
            stripe(sid * 640, 640)

        @pl.when(sid == NS - 1)
        def _():
            stripe((NS - 1) * 640, N - (NS - 1) * 640)

        pltpu.sync_copy(src_hbm.at[pl.ds(ebase, EW)], srcb.at[pl.ds(0, EW)])
        pltpu.sync_copy(dst_hbm.at[pl.ds(ebase, EW)], dstb.at[pl.ds(0, EW)])
        if EWP > EW:
            for j in range((EWP - EW) // L):
                srcb[pl.ds(EW + j * L, L)] = jnp.zeros((L,), jnp.int32)
                dstb[pl.ds(EW + j * L, L)] = jnp.zeros((L,), jnp.int32)

        plsc.subcore_barrier()

        def cchunk(off, n):
            h1 = pltpu.async_copy(st1.at[srcb.at[pl.ds(off, CH)]], mbuf, sem)
            h2 = pltpu.async_copy(st2.at[dstb.at[pl.ds(off, CH)]], nbuf, sem2)
            h3 = pltpu.async_copy(stn.at[srcb.at[pl.ds(off, CH)]], gn2, sem3)
            h1.wait()
            h2.wait()
            h3.wait()
            for j in range(n // L):
                m = jnp.exp(mbuf[pl.ds(j * L, L)] + nbuf[pl.ds(j * L, L)])
                nb = gn2[pl.ds(j * L, L)]
                safe = jnp.where(nb != 0.0, nb, jnp.ones((L,), jnp.float32))
                cb[pl.ds(off + j * L, L)] = jnp.where(
                    nb != 0.0, m / safe, jnp.zeros((L,), jnp.float32))

        def cbody(i, _):
            cchunk(i * CH, CH)
            return 0
        lax.fori_loop(0, nfull, cbody, 0)
        if tail:
            cchunk(nfull * CH, tail)

        def zh(i, _):
            hist[pl.ds(i * L, L)] = jnp.zeros((L,), jnp.int32)
            return 0
        lax.fori_loop(0, NBP, zh, 0)
        ones = jnp.ones((L,), jnp.int32)

        def hbody(i, _):
            s = srcb[pl.ds(i * L, L)]
            d = dstb[pl.ds(i * L, L)]
            plsc.addupdate_scatter(hist, [((d >> SHIFT) * 2) * L + lane], ones)
            plsc.addupdate_scatter(hist, [((s >> SHIFT) * 2 + 1) * L + lane],
                                   ones)
            return 0
        lax.fori_loop(0, nvec, hbody, 0)

        def sbody(b, run):
            h = hist[pl.ds(b * L, L)]
            inc = plsc.cumsum(h)
            curs[pl.ds(b * L, L)] = run + (inc - h)
            tot = jnp.sum(h)
            return run + (((tot + CH - 1) >> SHIFT) << SHIFT)
        lax.fori_loop(0, NBP, sbody, 0)

        def ebody(v, _):
            idx = (lane + v * L) * L
            cumb[pl.ds(v * L, L)] = plsc.load_gather(curs, [idx])
            return 0
        lax.fori_loop(0, NBP // L, ebody, 0)

        DUMT = SB * N
        def fbody(v, _):
            pos = v * L
            dums = DUMT + ((pos & 1023) + lane)
            r = pos & 15
            pada = ((DUMW + r) << 14) | (DUMW + r)
            stgi[pl.ds(pos, L)] = dums
            stga[pl.ds(pos, L)] = plsc.bitcast(
                jnp.full((L,), pada, jnp.int32), jnp.float32)
            return 0
        lax.fori_loop(0, PWP // L, fbody, 0)

        def pbody(i, _):
            s = srcb[pl.ds(i * L, L)]
            d = dstb[pl.ds(i * L, L)]
            c = cb[pl.ds(i * L, L)]
            ixw = (((d >> SHIFT) * 2) * L) + lane
            slot = plsc.load_gather(curs, [ixw])
            plsc.store_scatter(curs, [ixw], slot + 1)
            plsc.store_scatter(stgi, [slot], (d & (SB - 1)) * N + s)
            plsc.store_scatter(stga, [slot], c)
            ixr = (((s >> SHIFT) * 2 + 1) * L) + lane
            slot2 = plsc.load_gather(curs, [ixr])
            plsc.store_scatter(curs, [ixr], slot2 + 1)
            plsc.store_scatter(stgi, [slot2], (s & (SB - 1)) * N + d)
            plsc.store_scatter(stga, [slot2],
                               plsc.bitcast((s << 14) | d, jnp.float32))
            return 0
        lax.fori_loop(0, nvec, pbody, 0)

        pltpu.sync_copy(stgi, enti_hbm.at[pl.ds(wid * PWP, PWP)])
        pltpu.sync_copy(stga, enta_hbm.at[pl.ds(wid * PWP, PWP)])
        pltpu.sync_copy(cumb, cum_hbm.at[pl.ds(wid * NBP, NBP)])

    return k3


def _make_k4(N, NB, NBP, PWP, TSIZE, ZSPAN):
    NP16 = N + L
    ROWS = PWP // CH

    @functools.partial(
        pl.kernel, mesh=_mesh(),
        compiler_params=pltpu.CompilerParams(needs_layout_passes=False),
        out_type=jax.ShapeDtypeStruct((NW * N,), jnp.float32),
        scratch_types=[pltpu.VMEM((NW * NBP,), jnp.int32),
                       pltpu.VMEM((8, 1, CH), jnp.int32),
                       pltpu.VMEM((8, 1, CH), jnp.float32),
                       pltpu.VMEM((16, 1, CH), jnp.int32),
                       pltpu.VMEM((CH,), jnp.float32),
                       pltpu.VMEM((CH,), jnp.float32),
                       pltpu.VMEM((NP16,), jnp.float32),
                       pltpu.VMEM((2048,), jnp.float32),
                       pltpu.VMEM_SHARED((TSIZE,), jnp.float32),
                       pltpu.SemaphoreType.DMA,
                       pltpu.SemaphoreType.DMA,
                       pltpu.SemaphoreType.DMA,
                       pltpu.SemaphoreType.DMA,
                       pltpu.SemaphoreType.DMA,
                       pltpu.SemaphoreType.DMA,
                       pltpu.SemaphoreType.DMA,
                       pltpu.SemaphoreType.DMA],
    )
    def k4(enti_hbm, enta_hbm, cum_hbm, wpart_hbm,
           cums, ibuf, abuf, stash, g128, z128, wloc, zbuf, tmat,
           si0, si1, si2, si3, sa0, sa1, sa2, sa3):
        semi = [si0, si1, si2, si3]
        sema = [sa0, sa1, sa2, sa3]
        c = lax.axis_index("c")
        s = lax.axis_index("s")
        wid = s * NC + c
        pltpu.sync_copy(cum_hbm, cums)
        _zero_ref(zbuf, 2048 // L)
        _zero_ref(z128, CH // L)
        _zero_ref(wloc, NP16 // L)

        nz = ZSPAN // 2048
        zt = ZSPAN - nz * 2048

        def zb(i, _):
            pltpu.sync_copy(zbuf, tmat.at[pl.ds(s * ZSPAN + i * 2048, 2048)])
            return 0
        lax.fori_loop(0, nz, zb, 0)
        if zt:
            pltpu.sync_copy(zbuf.at[pl.ds(0, zt)],
                            tmat.at[pl.ds(s * ZSPAN + nz * 2048, zt)])
        plsc.subcore_barrier()

        def extract(k):
            v = plsc.load_gather(cums, [jnp.full((L,), k, jnp.int32)])
            return jnp.max(v)

        def ranges(bent):
            w0 = 2 * s
            w1 = 2 * s + 1
            lo0 = extract(w0 * NBP + bent)
            hi0 = extract(w0 * NBP + bent + 1)
            lo1 = extract(w1 * NBP + bent)
            hi1 = extract(w1 * NBP + bent + 1)
            return (w0 * ROWS + (lo0 >> SHIFT), (hi0 - lo0) >> SHIFT,
                    w1 * ROWS + (lo1 >> SHIFT), (hi1 - lo1) >> SHIFT)

        def hrow_of(rng):
            hr0, n0, hr1, n1 = rng

            def hrow(j):
                return jnp.where(j < n0, hr0 + j, hr1 + (j - n0))
            return hrow, n0 + n1

        def fire(rng, base, with_aux, s0=0):
            hrow, n = hrow_of(rng)
            for k in range(4):
                @pl.when(base + k < n)
                def _(k=k):
                    r = hrow(base + k)
                    pltpu.async_copy(enti_hbm.at[r], ibuf.at[s0 + k],
                                     semi[k])
                    if with_aux:
                        pltpu.async_copy(enta_hbm.at[r], abuf.at[s0 + k],
                                         sema[k])

        def merged(rng, with_aux, proc, prefired=False, s0=0):
            hrow, n = hrow_of(rng)

            def grp(g, _):
                base = g * 4
                pref = jnp.asarray(prefired, jnp.bool_)

                @pl.when(jnp.logical_or(g > 0, jnp.logical_not(pref)))
                def _():
                    fire(rng, base, with_aux, s0)
                for k in range(4):
                    @pl.when(base + k < n)
                    def _(k=k):
                        pltpu.make_async_copy(enti_hbm.at[0], ibuf.at[s0 + k],
                                              semi[k]).wait()
                        if with_aux:
                            pltpu.make_async_copy(enta_hbm.at[0],
                                                  abuf.at[s0 + k],
                                                  sema[k]).wait()
                        proc(s0 + k, base + k)
                return 0
            lax.fori_loop(0, (n + 3) >> 2, grp, 0)

        def proc_a(k, j):
            pltpu.sync_copy(abuf.at[k, 0], tmat.at[ibuf.at[k, 0]], add=True)

            @pl.when(j < 16)
            def _():
                for j2 in range(CH // L):
                    stash[j, 0, pl.ds(j2 * L, L)] = ibuf[k, 0,
                                                        pl.ds(j2 * L, L)]

        def proc_b(k, j):
            pltpu.sync_copy(tmat.at[ibuf.at[k, 0]], g128)
            for j2 in range(CH // L):
                co = g128[pl.ds(j2 * L, L)]
                ai = plsc.bitcast(abuf[k, 0, pl.ds(j2 * L, L)], jnp.int32)
                plsc.addupdate_scatter(wloc, [ai >> 14], co)
                plsc.addupdate_scatter(wloc, [ai & 0x3FFF], co)

        def phase_c(rng):
            hrow, n = hrow_of(rng)
            ns = jnp.minimum(n, 16)

            def cs(j, _):
                pltpu.sync_copy(z128, tmat.at[stash.at[j, 0]])
                return 0
            lax.fori_loop(0, ns, cs, 0)

            def ct(j, _):
                pltpu.sync_copy(enti_hbm.at[hrow(j)], ibuf.at[0])
                pltpu.sync_copy(z128, tmat.at[ibuf.at[0, 0]])
                return 0
            lax.fori_loop(ns, n, ct, 0)

        nq = (NB + 1) // 2 - c * (1 if NB % 2 else 0)

        def qloop(i, _):
            q = 2 * i + c
            r_w = ranges(2 * q)
            r_r = ranges(2 * q + 1)
            merged(r_w, True, proc_a, prefired=i > 0)
            fire(r_r, 0, True, s0=4)
            plsc.subcore_barrier()
            merged(r_r, True, proc_b, prefired=True, s0=4)
            plsc.subcore_barrier()
            phase_c(r_w)

            @pl.when(i + 1 < nq)
            def _():
                fire(ranges(2 * (2 * (i + 1) + c)), 0, True)
            plsc.subcore_barrier()
            return 0
        lax.fori_loop(0, nq, qloop, 0)
        pltpu.sync_copy(wloc.at[pl.ds(0, N)], wpart_hbm.at[pl.ds(wid * N, N)])

    return k4


def kernel(node_features, edge_list, kernel, attention_kernel, bias):
    N, F = node_features.shape
    D = kernel.shape[1]
    E = edge_list.shape[0]

    EW = E // NW
    assert EW * NW == E and EW % L == 0
    nfull, tail = divmod(EW, CH)
    NB = (N + SB - 1) // SB
    NBP = ((2 * NB + L - 1) // L) * L
    PWP = 2 * EW + NBP * (CH - 1)
    PWP = ((PWP + CH - 1) // CH) * CH
    TSIZE = SB * N + 1024
    ZSPAN = TSIZE // NS
    assert ZSPAN * NS == TSIZE and ZSPAN % 8 == 0
    NPAD = ((N + 511) // 512) * 512

    src = edge_list[:, 0]
    dst = edge_list[:, 1]
    ak2 = jnp.pad(attention_kernel.reshape(2, D).T, ((0, 0), (0, D - 2)))
    xpad = jnp.pad(node_features, ((0, NPAD - N), (0, 0)))

    keys, s12 = pl.pallas_call(
        _k0_body,
        grid=(NPAD // 512,),
        in_specs=[pl.BlockSpec((512, F), lambda i: (i, 0)),
                  pl.BlockSpec((F, D), lambda i: (0, 0)),
                  pl.BlockSpec((D, D), lambda i: (0, 0))],
        out_specs=[pl.BlockSpec((512, D), lambda i: (i, 0)),
                   pl.BlockSpec((512, D), lambda i: (i, 0))],
        out_shape=[jax.ShapeDtypeStruct((NPAD, D), jnp.float32),
                   jax.ShapeDtypeStruct((NPAD, D), jnp.float32)],
    )(xpad, kernel, ak2)

    s1 = s12[:N, 0]
    s2 = s12[:N, 1]

    parts = _make_k1(N, E, EW, nfull, tail)(src, dst, s1, s2)
    enti, enta, cum = _make_k3(N, E, EW, nfull, tail, NBP, PWP, N)(
        src, dst, s1, s2, parts)
    ROWS = PWP // CH
    wparts = _make_k4(N, NB, NBP, PWP, TSIZE, ZSPAN)(
        enti.reshape(NW * ROWS, 1, CH), enta.reshape(NW * ROWS, 1, CH), cum)

    wpad = jnp.pad(wparts.reshape(NW, N), ((0, 0), (0, NPAD - N)))
    out = pl.pallas_call(
        _k5_body,
        grid=(NPAD // 512,),
        in_specs=[pl.BlockSpec((512, D), lambda i: (i, 0)),
                  pl.BlockSpec((NW, 512), lambda i: (0, i)),
                  pl.BlockSpec((1, D), lambda i: (0, 0))],
        out_specs=pl.BlockSpec((512, D), lambda i: (i, 0)),
        out_shape=jax.ShapeDtypeStruct((NPAD, D), jnp.float32),
    )(keys, wpad, bias.reshape(1, D))
    return out[:N]


# speedup vs baseline: 30.0411x; 1.0861x over previous
import functools

import jax
import jax.numpy as jnp
from jax import lax
from jax.experimental import pallas as pl
from jax.experimental.pallas import tpu as pltpu
from jax.experimental.pallas import tpu_sc as plsc

NC = 2
NS = 16
L = 16
NW = NC * NS

CH = 128
SHIFT = 7
SB = 160


def _blk(x):
    return (x * 52429) >> 23


def _mesh():
    return plsc.VectorSubcoreMesh(core_axis_name="c", subcore_axis_name="s")


def _lane():
    return lax.broadcasted_iota(jnp.int32, (L,), 0)


def _zero_ref(ref, nvec):
    def b(i, _):
        ref[pl.ds(i * L, L)] = jnp.zeros((L,), jnp.float32)
        return 0
    lax.fori_loop(0, nvec, b, 0)


def _k0_body(x_ref, w_ref, a2_ref, keys_ref, s_ref):
    k = jnp.dot(x_ref[...], w_ref[...], preferred_element_type=jnp.float32)
    keys_ref[...] = k
    s_ref[...] = jnp.dot(k, a2_ref[...], preferred_element_type=jnp.float32)


def _k5_body(keys_ref, wp_ref, b_ref, o_ref):
    w = jnp.sum(wp_ref[...], axis=0)
    o_ref[...] = jnp.maximum(keys_ref[...] * w[:, None] + b_ref[...], 0.0)


def _make_k1(N, E, EW, nfull, tail):
    EWP = EW + (CH - tail if tail else 0)

    @functools.partial(
        pl.kernel, mesh=_mesh(),
        compiler_params=pltpu.CompilerParams(needs_layout_passes=False),
        out_type=jax.ShapeDtypeStruct((NW * N,), jnp.float32),
        scratch_types=[pltpu.VMEM((EWP,), jnp.int32),
                       pltpu.VMEM((EWP,), jnp.int32),
                       pltpu.VMEM((CH,), jnp.float32),
                       pltpu.VMEM((CH,), jnp.float32),
                       pltpu.VMEM((N,), jnp.float32),
                       pltpu.VMEM_SHARED((N,), jnp.float32),
                       pltpu.VMEM_SHARED((N,), jnp.float32),
                       pltpu.SemaphoreType.DMA,
                       pltpu.SemaphoreType.DMA],
    )
    def k1(src_hbm, dst_hbm, s1_hbm, s2_hbm, part_hbm,
           srcb, dstb, g1, g2, ngh, st1, st2, sem, sem2):
        wid = lax.axis_index("s") * NC + lax.axis_index("c")
        ebase = wid * EW

        @pl.when(lax.axis_index("s") == 0)
        def _():
            pltpu.sync_copy(s1_hbm, st1)
            pltpu.sync_copy(s2_hbm, st2)
        pltpu.sync_copy(src_hbm.at[pl.ds(ebase, EW)], srcb.at[pl.ds(0, EW)])
        pltpu.sync_copy(dst_hbm.at[pl.ds(ebase, EW)], dstb.at[pl.ds(0, EW)])
        if EWP > EW:
            for j in range((EWP - EW) // L):
                srcb[pl.ds(EW + j * L, L)] = jnp.zeros((L,), jnp.int32)
                dstb[pl.ds(EW + j * L, L)] = jnp.zeros((L,), jnp.int32)
        _zero_ref(ngh, N // L)
        plsc.subcore_barrier()

        def chunk(off, n):
            h1 = pltpu.async_copy(st1.at[srcb.at[pl.ds(off, CH)]], g1, sem)
            h2 = pltpu.async_copy(st2.at[dstb.at[pl.ds(off, CH)]], g2, sem2)
            h1.wait()
            h2.wait()
            for j in range(n // L):
                sv = srcb[pl.ds(off + j * L, L)]
                m = jnp.exp(g1[pl.ds(j * L, L)] + g2[pl.ds(j * L, L)])
                plsc.addupdate_scatter(ngh, [sv], m)

        def body(i, _):
            chunk(i * CH, CH)
            return 0
        lax.fori_loop(0, nfull, body, 0)
        if tail:
            chunk(nfull * CH, tail)
        pltpu.sync_copy(ngh, part_hbm.at[pl.ds(wid * N, N)])

    assert nfull > 0
    return k1


def _make_k3(N, E, EW, nfull, tail, NBP, PWP, DUMW):
    EWP = EW + (CH - tail if tail else 0)
    nvec = EW // L

    @functools.partial(
        pl.kernel, mesh=_mesh(),
        compiler_params=pltpu.CompilerParams(needs_layout_passes=False),
        out_type=[jax.ShapeDtypeStruct((NW * PWP,), jnp.int32),
                  jax.ShapeDtypeStruct((NW * PWP,), jnp.float32),
                  jax.ShapeDtypeStruct((NW * NBP,), jnp.int32)],
        scratch_types=[pltpu.VMEM((EWP,), jnp.int32),
                       pltpu.VMEM((EWP,), jnp.int32),
                       pltpu.VMEM((EW,), jnp.float32),
                       pltpu.VMEM((CH,), jnp.float32),
                       pltpu.VMEM((CH,), jnp.float32),
                       pltpu.VMEM((NBP * L,), jnp.int32),
                       pltpu.VMEM((NBP * L,), jnp.int32),
                       pltpu.VMEM((NBP,), jnp.int32),
                       pltpu.VMEM((PWP,), jnp.int32),
                       pltpu.VMEM((PWP,), jnp.float32),
                       pltpu.VMEM((4 * 640,), jnp.float32),
                       pltpu.VMEM((640,), jnp.float32),
                       pltpu.VMEM((CH,), jnp.float32),
                       pltpu.VMEM_SHARED((N,), jnp.float32),
                       pltpu.VMEM_SHARED((N,), jnp.float32),
                       pltpu.VMEM_SHARED((N,), jnp.float32),
                       pltpu.SemaphoreType.DMA,
                       pltpu.SemaphoreType.DMA,
                       pltpu.SemaphoreType.DMA,
                       pltpu.SemaphoreType.DMA],
    )
    def k3(src_hbm, dst_hbm, s1_hbm, s2_hbm, part_hbm, enti_hbm, enta_hbm,
           cum_hbm, srcb, dstb, cb, nbuf, mbuf, hist, curs, cumb, stgi, stga,
           pb, ab, gn2, stn, st1, st2, sem, sem2, sem3, sem4):
        sid = lax.axis_index("s")
        wid = sid * NC + lax.axis_index("c")
        ebase = wid * EW
        lane = _lane()

        @pl.when(sid == 0)
        def _():
            pltpu.sync_copy(s1_hbm, st1)
            pltpu.sync_copy(s2_hbm, st2)

        def stripe(col0, ln):
            sems4 = [sem, sem2, sem3, sem4]
            for j in range(ln // L):
                ab[pl.ds(j * L, L)] = jnp.zeros((L,), jnp.float32)

            def fire_w(w):
                pltpu.async_copy(part_hbm.at[pl.ds(w * N + col0, ln)],
                                 pb.at[pl.ds((w % 4) * 640, ln)], sems4[w % 4])

            def drain_w(w):
                pltpu.make_async_copy(part_hbm.at[pl.ds(0, ln)],
                                      pb.at[pl.ds((w % 4) * 640, ln)],
                                      sems4[w % 4]).wait()
                for j in range(ln // L):
                    ab[pl.ds(j * L, L)] = (ab[pl.ds(j * L, L)] +
                                           pb[pl.ds((w % 4) * 640 + j * L, L)])
            for w in range(4):
                fire_w(w)
            for w in range(NW):
                drain_w(w)
                if w + 4 < NW:
                    fire_w(w + 4)
            pltpu.sync_copy(ab.at[pl.ds(0, ln)], stn.at[pl.ds(col0, ln)])

        @pl.when(sid < NS - 1)
        def _():
            stripe(sid * 640, 640)

        @pl.when(sid == NS - 1)
        def _():
            stripe((NS - 1) * 640, N - (NS - 1) * 640)

        pltpu.sync_copy(src_hbm.at[pl.ds(ebase, EW)], srcb.at[pl.ds(0, EW)])
        pltpu.sync_copy(dst_hbm.at[pl.ds(ebase, EW)], dstb.at[pl.ds(0, EW)])
        if EWP > EW:
            for j in range((EWP - EW) // L):
                srcb[pl.ds(EW + j * L, L)] = jnp.zeros((L,), jnp.int32)
                dstb[pl.ds(EW + j * L, L)] = jnp.zeros((L,), jnp.int32)

        plsc.subcore_barrier()

        def cchunk(off, n):
            h1 = pltpu.async_copy(st1.at[srcb.at[pl.ds(off, CH)]], mbuf, sem)
            h2 = pltpu.async_copy(st2.at[dstb.at[pl.ds(off, CH)]], nbuf, sem2)
            h3 = pltpu.async_copy(stn.at[srcb.at[pl.ds(off, CH)]], gn2, sem3)
            h1.wait()
            h2.wait()
            h3.wait()
            for j in range(n // L):
                m = jnp.exp(mbuf[pl.ds(j * L, L)] + nbuf[pl.ds(j * L, L)])
                nb = gn2[pl.ds(j * L, L)]
                safe = jnp.where(nb != 0.0, nb, jnp.ones((L,), jnp.float32))
                cb[pl.ds(off + j * L, L)] = jnp.where(
                    nb != 0.0, m / safe, jnp.zeros((L,), jnp.float32))

        def cbody(i, _):
            cchunk(i * CH, CH)
            return 0
        lax.fori_loop(0, nfull, cbody, 0)
        if tail:
            cchunk(nfull * CH, tail)

        def zh(i, _):
            hist[pl.ds(i * L, L)] = jnp.zeros((L,), jnp.int32)
            return 0
        lax.fori_loop(0, NBP, zh, 0)
        ones = jnp.ones((L,), jnp.int32)

        def hbody(i, _):
            s = srcb[pl.ds(i * L, L)]
            d = dstb[pl.ds(i * L, L)]
            plsc.addupdate_scatter(hist, [(_blk(d) * 2) * L + lane], ones)
            plsc.addupdate_scatter(hist, [(_blk(s) * 2 + 1) * L + lane],
                                   ones)
            return 0
        lax.fori_loop(0, nvec, hbody, 0)

        def sbody(b, run):
            h = hist[pl.ds(b * L, L)]
            inc = plsc.cumsum(h)
            curs[pl.ds(b * L, L)] = run + (inc - h)
            tot = jnp.sum(h)
            return run + (((tot + CH - 1) >> SHIFT) << SHIFT)
        lax.fori_loop(0, NBP, sbody, 0)

        def ebody(v, _):
            idx = (lane + v * L) * L
            cumb[pl.ds(v * L, L)] = plsc.load_gather(curs, [idx])
            return 0
        lax.fori_loop(0, NBP // L, ebody, 0)

        DUMT = SB * N
        def fbody(v, _):
            pos = v * L
            dums = DUMT + ((pos & 1023) + lane)
            r = pos & 15
            pada = ((DUMW + r) << 14) | (DUMW + r)
            stgi[pl.ds(pos, L)] = dums
            stga[pl.ds(pos, L)] = plsc.bitcast(
                jnp.full((L,), pada, jnp.int32), jnp.float32)
            return 0
        lax.fori_loop(0, PWP // L, fbody, 0)

        def pbody(i, _):
            s = srcb[pl.ds(i * L, L)]
            d = dstb[pl.ds(i * L, L)]
            c = cb[pl.ds(i * L, L)]
            bd = _blk(d)
            bs = _blk(s)
            ixw = ((bd * 2) * L) + lane
            slot = plsc.load_gather(curs, [ixw])
            plsc.store_scatter(curs, [ixw], slot + 1)
            plsc.store_scatter(stgi, [slot], (d - bd * SB) * N + s)
            plsc.store_scatter(stga, [slot], c)
            ixr = ((bs * 2 + 1) * L) + lane
            slot2 = plsc.load_gather(curs, [ixr])
            plsc.store_scatter(curs, [ixr], slot2 + 1)
            plsc.store_scatter(stgi, [slot2], (s - bs * SB) * N + d)
            plsc.store_scatter(stga, [slot2],
                               plsc.bitcast((s << 14) | d, jnp.float32))
            return 0
        lax.fori_loop(0, nvec, pbody, 0)

        pltpu.sync_copy(stgi, enti_hbm.at[pl.ds(wid * PWP, PWP)])
        pltpu.sync_copy(stga, enta_hbm.at[pl.ds(wid * PWP, PWP)])
        pltpu.sync_copy(cumb, cum_hbm.at[pl.ds(wid * NBP, NBP)])

    return k3


def _make_k4(N, NB, NBP, PWP, TSIZE, ZSPAN):
    NP16 = N + L
    ROWS = PWP // CH

    @functools.partial(
        pl.kernel, mesh=_mesh(),
        compiler_params=pltpu.CompilerParams(needs_layout_passes=False),
        out_type=jax.ShapeDtypeStruct((NW * N,), jnp.float32),
        scratch_types=[pltpu.VMEM((NW * NBP,), jnp.int32),
                       pltpu.VMEM((8, 1, CH), jnp.int32),
                       pltpu.VMEM((8, 1, CH), jnp.float32),
                       pltpu.VMEM((16, 1, CH), jnp.int32),
                       pltpu.VMEM((CH,), jnp.float32),
                       pltpu.VMEM((CH,), jnp.float32),
                       pltpu.VMEM((NP16,), jnp.float32),
                       pltpu.VMEM((2048,), jnp.float32),
                       pltpu.VMEM_SHARED((TSIZE,), jnp.float32),
                       pltpu.SemaphoreType.DMA,
                       pltpu.SemaphoreType.DMA,
                       pltpu.SemaphoreType.DMA,
                       pltpu.SemaphoreType.DMA,
                       pltpu.SemaphoreType.DMA,
                       pltpu.SemaphoreType.DMA,
                       pltpu.SemaphoreType.DMA,
                       pltpu.SemaphoreType.DMA],
    )
    def k4(enti_hbm, enta_hbm, cum_hbm, wpart_hbm,
           cums, ibuf, abuf, stash, g128, z128, wloc, zbuf, tmat,
           si0, si1, si2, si3, sa0, sa1, sa2, sa3):
        semi = [si0, si1, si2, si3]
        sema = [sa0, sa1, sa2, sa3]
        c = lax.axis_index("c")
        s = lax.axis_index("s")
        wid = s * NC + c
        pltpu.sync_copy(cum_hbm, cums)
        _zero_ref(zbuf, 2048 // L)
        _zero_ref(z128, CH // L)
        _zero_ref(wloc, NP16 // L)

        nz = ZSPAN // 2048
        zt = ZSPAN - nz * 2048

        def zb(i, _):
            pltpu.sync_copy(zbuf, tmat.at[pl.ds(s * ZSPAN + i * 2048, 2048)])
            return 0
        lax.fori_loop(0, nz, zb, 0)
        if zt:
            pltpu.sync_copy(zbuf.at[pl.ds(0, zt)],
                            tmat.at[pl.ds(s * ZSPAN + nz * 2048, zt)])
        plsc.subcore_barrier()

        def extract(k):
            v = plsc.load_gather(cums, [jnp.full((L,), k, jnp.int32)])
            return jnp.max(v)

        def ranges(bent):
            w0 = 2 * s
            w1 = 2 * s + 1
            lo0 = extract(w0 * NBP + bent)
            hi0 = extract(w0 * NBP + bent + 1)
            lo1 = extract(w1 * NBP + bent)
            hi1 = extract(w1 * NBP + bent + 1)
            return (w0 * ROWS + (lo0 >> SHIFT), (hi0 - lo0) >> SHIFT,
                    w1 * ROWS + (lo1 >> SHIFT), (hi1 - lo1) >> SHIFT)

        def hrow_of(rng):
            hr0, n0, hr1, n1 = rng

            def hrow(j):
                return jnp.where(j < n0, hr0 + j, hr1 + (j - n0))
            return hrow, n0 + n1

        def fire(rng, base, with_aux, s0=0):
            hrow, n = hrow_of(rng)
            for k in range(4):
                @pl.when(base + k < n)
                def _(k=k):
                    r = hrow(base + k)
                    pltpu.async_copy(enti_hbm.at[r], ibuf.at[s0 + k],
                                     semi[k])
                    if with_aux:
                        pltpu.async_copy(enta_hbm.at[r], abuf.at[s0 + k],
                                         sema[k])

        def merged(rng, with_aux, proc, prefired=False, s0=0):
            hrow, n = hrow_of(rng)

            def grp(g, _):
                base = g * 4
                pref = jnp.asarray(prefired, jnp.bool_)

                @pl.when(jnp.logical_or(g > 0, jnp.logical_not(pref)))
                def _():
                    fire(rng, base, with_aux, s0)
                for k in range(4):
                    @pl.when(base + k < n)
                    def _(k=k):
                        pltpu.make_async_copy(enti_hbm.at[0], ibuf.at[s0 + k],
                                              semi[k]).wait()
                        if with_aux:
                            pltpu.make_async_copy(enta_hbm.at[0],
                                                  abuf.at[s0 + k],
                                                  sema[k]).wait()
                        proc(s0 + k, base + k)
                return 0
            lax.fori_loop(0, (n + 3) >> 2, grp, 0)

        def proc_a(k, j):
            pltpu.sync_copy(abuf.at[k, 0], tmat.at[ibuf.at[k, 0]], add=True)

            @pl.when(j < 16)
            def _():
                for j2 in range(CH // L):
                    stash[j, 0, pl.ds(j2 * L, L)] = ibuf[k, 0,
                                                        pl.ds(j2 * L, L)]

        def proc_b(k, j):
            pltpu.sync_copy(tmat.at[ibuf.at[k, 0]], g128)
            for j2 in range(CH // L):
                co = g128[pl.ds(j2 * L, L)]
                ai = plsc.bitcast(abuf[k, 0, pl.ds(j2 * L, L)], jnp.int32)
                plsc.addupdate_scatter(wloc, [ai >> 14], co)
                plsc.addupdate_scatter(wloc, [ai & 0x3FFF], co)

        def phase_c(rng):
            hrow, n = hrow_of(rng)
            ns = jnp.minimum(n, 16)

            def cs(j, _):
                pltpu.sync_copy(z128, tmat.at[stash.at[j, 0]])
                return 0
            lax.fori_loop(0, ns, cs, 0)

            def ct(j, _):
                pltpu.sync_copy(enti_hbm.at[hrow(j)], ibuf.at[0])
                pltpu.sync_copy(z128, tmat.at[ibuf.at[0, 0]])
                return 0
            lax.fori_loop(ns, n, ct, 0)

        nq = (NB + 1) // 2 - c * (1 if NB % 2 else 0)

        def qloop(i, _):
            q = 2 * i + c
            r_w = ranges(2 * q)
            r_r = ranges(2 * q + 1)
            merged(r_w, True, proc_a, prefired=i > 0)
            fire(r_r, 0, True, s0=4)
            plsc.subcore_barrier()
            merged(r_r, True, proc_b, prefired=True, s0=4)
            plsc.subcore_barrier()
            phase_c(r_w)

            @pl.when(i + 1 < nq)
            def _():
                fire(ranges(2 * (2 * (i + 1) + c)), 0, True)
            plsc.subcore_barrier()
            return 0
        lax.fori_loop(0, nq, qloop, 0)
        pltpu.sync_copy(wloc.at[pl.ds(0, N)], wpart_hbm.at[pl.ds(wid * N, N)])

    return k4


def kernel(node_features, edge_list, kernel, attention_kernel, bias):
    N, F = node_features.shape
    D = kernel.shape[1]
    E = edge_list.shape[0]

    EW = E // NW
    assert EW * NW == E and EW % L == 0
    nfull, tail = divmod(EW, CH)
    NB = (N + SB - 1) // SB
    NBP = ((2 * NB + L - 1) // L) * L
    PWP = 2 * EW + NBP * (CH - 1)
    PWP = ((PWP + CH - 1) // CH) * CH
    TSIZE = SB * N + 1024
    ZSPAN = TSIZE // NS
    assert ZSPAN * NS == TSIZE and ZSPAN % 8 == 0
    NPAD = ((N + 511) // 512) * 512

    src = edge_list[:, 0]
    dst = edge_list[:, 1]
    ak2 = jnp.pad(attention_kernel.reshape(2, D).T, ((0, 0), (0, D - 2)))
    xpad = jnp.pad(node_features, ((0, NPAD - N), (0, 0)))

    keys, s12 = pl.pallas_call(
        _k0_body,
        grid=(NPAD // 512,),
        in_specs=[pl.BlockSpec((512, F), lambda i: (i, 0)),
                  pl.BlockSpec((F, D), lambda i: (0, 0)),
                  pl.BlockSpec((D, D), lambda i: (0, 0))],
        out_specs=[pl.BlockSpec((512, D), lambda i: (i, 0)),
                   pl.BlockSpec((512, D), lambda i: (i, 0))],
        out_shape=[jax.ShapeDtypeStruct((NPAD, D), jnp.float32),
                   jax.ShapeDtypeStruct((NPAD, D), jnp.float32)],
    )(xpad, kernel, ak2)

    s1 = s12[:N, 0]
    s2 = s12[:N, 1]

    parts = _make_k1(N, E, EW, nfull, tail)(src, dst, s1, s2)
    enti, enta, cum = _make_k3(N, E, EW, nfull, tail, NBP, PWP, N)(
        src, dst, s1, s2, parts)
    ROWS = PWP // CH
    wparts = _make_k4(N, NB, NBP, PWP, TSIZE, ZSPAN)(
        enti.reshape(NW * ROWS, 1, CH), enta.reshape(NW * ROWS, 1, CH), cum)

    wpad = jnp.pad(wparts.reshape(NW, N), ((0, 0), (0, NPAD - N)))
    out = pl.pallas_call(
        _k5_body,
        grid=(NPAD // 512,),
        in_specs=[pl.BlockSpec((512, D), lambda i: (i, 0)),
                  pl.BlockSpec((NW, 512), lambda i: (0, i)),
                  pl.BlockSpec((1, D), lambda i: (0, 0))],
        out_specs=pl.BlockSpec((512, D), lambda i: (i, 0)),
        out_shape=jax.ShapeDtypeStruct((NPAD, D), jnp.float32),
    )(keys, wpad, bias.reshape(1, D))
    return out[:N]


# speedup vs baseline: 31.1014x; 1.0353x over previous
import functools

import jax
import jax.numpy as jnp
from jax import lax
from jax.experimental import pallas as pl
from jax.experimental.pallas import tpu as pltpu
from jax.experimental.pallas import tpu_sc as plsc

NC = 2
NS = 16
L = 16
NW = NC * NS

CH = 128
SHIFT = 7
SB = 160


def _blk(x):
    return (x * 52429) >> 23


def _mesh():
    return plsc.VectorSubcoreMesh(core_axis_name="c", subcore_axis_name="s")


def _lane():
    return lax.broadcasted_iota(jnp.int32, (L,), 0)


def _zero_ref(ref, nvec):
    def b(i, _):
        ref[pl.ds(i * L, L)] = jnp.zeros((L,), jnp.float32)
        return 0
    lax.fori_loop(0, nvec, b, 0)


def _k0_body(x_ref, w_ref, a2_ref, keys_ref, s_ref):
    k = jnp.dot(x_ref[...], w_ref[...], preferred_element_type=jnp.float32)
    keys_ref[...] = k
    s_ref[...] = jnp.dot(k, a2_ref[...], preferred_element_type=jnp.float32)


def _k5_body(keys_ref, wp_ref, b_ref, o_ref):
    w = jnp.sum(wp_ref[...], axis=0)
    o_ref[...] = jnp.maximum(keys_ref[...] * w[:, None] + b_ref[...], 0.0)


def _make_k1(N, E, EW, nfull, tail):
    EWP = EW + (CH - tail if tail else 0)

    @functools.partial(
        pl.kernel, mesh=_mesh(),
        compiler_params=pltpu.CompilerParams(needs_layout_passes=False),
        out_type=jax.ShapeDtypeStruct((NW * N,), jnp.float32),
        scratch_types=[pltpu.VMEM((EWP,), jnp.int32),
                       pltpu.VMEM((EWP,), jnp.int32),
                       pltpu.VMEM((CH,), jnp.float32),
                       pltpu.VMEM((CH,), jnp.float32),
                       pltpu.VMEM((CH,), jnp.float32),
                       pltpu.VMEM((CH,), jnp.float32),
                       pltpu.VMEM((N,), jnp.float32),
                       pltpu.VMEM_SHARED((N,), jnp.float32),
                       pltpu.VMEM_SHARED((N,), jnp.float32),
                       pltpu.SemaphoreType.DMA,
                       pltpu.SemaphoreType.DMA,
                       pltpu.SemaphoreType.DMA,
                       pltpu.SemaphoreType.DMA],
    )
    def k1(src_hbm, dst_hbm, s1_hbm, s2_hbm, part_hbm,
           srcb, dstb, g1a, g2a, g1b, g2b, ngh, st1, st2,
           s1a, s2a, s1b, s2b):
        wid = lax.axis_index("s") * NC + lax.axis_index("c")
        ebase = wid * EW

        @pl.when(lax.axis_index("s") == 0)
        def _():
            pltpu.sync_copy(s1_hbm, st1)
            pltpu.sync_copy(s2_hbm, st2)
        pltpu.sync_copy(src_hbm.at[pl.ds(ebase, EW)], srcb.at[pl.ds(0, EW)])
        pltpu.sync_copy(dst_hbm.at[pl.ds(ebase, EW)], dstb.at[pl.ds(0, EW)])
        if EWP > EW:
            for j in range((EWP - EW) // L):
                srcb[pl.ds(EW + j * L, L)] = jnp.zeros((L,), jnp.int32)
                dstb[pl.ds(EW + j * L, L)] = jnp.zeros((L,), jnp.int32)
        _zero_ref(ngh, N // L)
        plsc.subcore_barrier()

        nch = nfull + (1 if tail else 0)
        bufs = ((g1a, g2a, s1a, s2a), (g1b, g2b, s1b, s2b))

        def fire(ci, k):
            b1, b2, x1, x2 = bufs[k]
            pltpu.async_copy(st1.at[srcb.at[pl.ds(ci * CH, CH)]], b1, x1)
            pltpu.async_copy(st2.at[dstb.at[pl.ds(ci * CH, CH)]], b2, x2)

        def proc(ci, k, n):
            b1, b2, x1, x2 = bufs[k]
            pltpu.make_async_copy(st1.at[pl.ds(0, CH)], b1, x1).wait()
            pltpu.make_async_copy(st1.at[pl.ds(0, CH)], b2, x2).wait()
            for j in range(n // L):
                sv = srcb[pl.ds(ci * CH + j * L, L)]
                m = jnp.exp(b1[pl.ds(j * L, L)] + b2[pl.ds(j * L, L)])
                plsc.addupdate_scatter(ngh, [sv], m)

        fire(0, 0)

        def body(p, _):
            @pl.when(2 * p + 1 < nch)
            def _():
                fire(2 * p + 1, 1)
            proc(2 * p, 0, CH)

            @pl.when(2 * p + 2 < nch)
            def _():
                fire(2 * p + 2, 0)

            @pl.when(2 * p + 1 < nfull)
            def _():
                proc(2 * p + 1, 1, CH)
            return 0
        lax.fori_loop(0, nfull // 2, body, 0)
        fired_max = nfull if nfull % 2 == 0 else nfull - 1
        for ci in range(2 * (nfull // 2), nch):
            if ci > fired_max:
                fire(ci, ci % 2)
            proc(ci, ci % 2, CH if ci < nfull else tail)
        pltpu.sync_copy(ngh, part_hbm.at[pl.ds(wid * N, N)])

    assert nfull > 0
    return k1


def _make_k3(N, E, EW, nfull, tail, NBP, PWP, DUMW):
    EWP = EW + (CH - tail if tail else 0)
    nvec = EW // L

    @functools.partial(
        pl.kernel, mesh=_mesh(),
        compiler_params=pltpu.CompilerParams(needs_layout_passes=False),
        out_type=[jax.ShapeDtypeStruct((NW * PWP,), jnp.int32),
                  jax.ShapeDtypeStruct((NW * PWP,), jnp.float32),
                  jax.ShapeDtypeStruct((NW * NBP,), jnp.int32)],
        scratch_types=[pltpu.VMEM((EWP,), jnp.int32),
                       pltpu.VMEM((EWP,), jnp.int32),
                       pltpu.VMEM((EW,), jnp.float32),
                       pltpu.VMEM((CH,), jnp.float32),
                       pltpu.VMEM((CH,), jnp.float32),
                       pltpu.VMEM((NBP * L,), jnp.int32),
                       pltpu.VMEM((NBP * L,), jnp.int32),
                       pltpu.VMEM((NBP,), jnp.int32),
                       pltpu.VMEM((PWP,), jnp.int32),
                       pltpu.VMEM((PWP,), jnp.float32),
                       pltpu.VMEM((4 * 640,), jnp.float32),
                       pltpu.VMEM((640,), jnp.float32),
                       pltpu.VMEM((CH,), jnp.float32),
                       pltpu.VMEM_SHARED((N,), jnp.float32),
                       pltpu.VMEM_SHARED((N,), jnp.float32),
                       pltpu.VMEM_SHARED((N,), jnp.float32),
                       pltpu.SemaphoreType.DMA,
                       pltpu.SemaphoreType.DMA,
                       pltpu.SemaphoreType.DMA,
                       pltpu.SemaphoreType.DMA],
    )
    def k3(src_hbm, dst_hbm, s1_hbm, s2_hbm, part_hbm, enti_hbm, enta_hbm,
           cum_hbm, srcb, dstb, cb, nbuf, mbuf, hist, curs, cumb, stgi, stga,
           pb, ab, gn2, stn, st1, st2, sem, sem2, sem3, sem4):
        sid = lax.axis_index("s")
        wid = sid * NC + lax.axis_index("c")
        ebase = wid * EW
        lane = _lane()

        @pl.when(sid == 0)
        def _():
            pltpu.sync_copy(s1_hbm, st1)
            pltpu.sync_copy(s2_hbm, st2)

        def stripe(col0, ln):
            sems4 = [sem, sem2, sem3, sem4]
            for j in range(ln // L):
                ab[pl.ds(j * L, L)] = jnp.zeros((L,), jnp.float32)

            def fire_w(w):
                pltpu.async_copy(part_hbm.at[pl.ds(w * N + col0, ln)],
                                 pb.at[pl.ds((w % 4) * 640, ln)], sems4[w % 4])

            def drain_w(w):
                pltpu.make_async_copy(part_hbm.at[pl.ds(0, ln)],
                                      pb.at[pl.ds((w % 4) * 640, ln)],
                                      sems4[w % 4]).wait()
                for j in range(ln // L):
                    ab[pl.ds(j * L, L)] = (ab[pl.ds(j * L, L)] +
                                           pb[pl.ds((w % 4) * 640 + j * L, L)])
            for w in range(4):
                fire_w(w)
            for w in range(NW):
                drain_w(w)
                if w + 4 < NW:
                    fire_w(w + 4)
            pltpu.sync_copy(ab.at[pl.ds(0, ln)], stn.at[pl.ds(col0, ln)])

        @pl.when(sid < NS - 1)
        def _():
            stripe(sid * 640, 640)

        @pl.when(sid == NS - 1)
        def _():
            stripe((NS - 1) * 640, N - (NS - 1) * 640)

        pltpu.sync_copy(src_hbm.at[pl.ds(ebase, EW)], srcb.at[pl.ds(0, EW)])
        pltpu.sync_copy(dst_hbm.at[pl.ds(ebase, EW)], dstb.at[pl.ds(0, EW)])
        if EWP > EW:
            for j in range((EWP - EW) // L):
                srcb[pl.ds(EW + j * L, L)] = jnp.zeros((L,), jnp.int32)
                dstb[pl.ds(EW + j * L, L)] = jnp.zeros((L,), jnp.int32)

        plsc.subcore_barrier()

        def cchunk(off, n):
            h1 = pltpu.async_copy(st1.at[srcb.at[pl.ds(off, CH)]], mbuf, sem)
            h2 = pltpu.async_copy(st2.at[dstb.at[pl.ds(off, CH)]], nbuf, sem2)
            h3 = pltpu.async_copy(stn.at[srcb.at[pl.ds(off, CH)]], gn2, sem3)
            h1.wait()
            h2.wait()
            h3.wait()
            for j in range(n // L):
                m = jnp.exp(mbuf[pl.ds(j * L, L)] + nbuf[pl.ds(j * L, L)])
                nb = gn2[pl.ds(j * L, L)]
                safe = jnp.where(nb != 0.0, nb, jnp.ones((L,), jnp.float32))
                cb[pl.ds(off + j * L, L)] = jnp.where(
                    nb != 0.0, m / safe, jnp.zeros((L,), jnp.float32))

        def cbody(i, _):
            cchunk(i * CH, CH)
            return 0
        lax.fori_loop(0, nfull, cbody, 0)
        if tail:
            cchunk(nfull * CH, tail)

        def zh(i, _):
            hist[pl.ds(i * L, L)] = jnp.zeros((L,), jnp.int32)
            return 0
        lax.fori_loop(0, NBP, zh, 0)
        ones = jnp.ones((L,), jnp.int32)

        def hbody(i, _):
            s = srcb[pl.ds(i * L, L)]
            d = dstb[pl.ds(i * L, L)]
            plsc.addupdate_scatter(hist, [(_blk(d) * 2) * L + lane], ones)
            plsc.addupdate_scatter(hist, [(_blk(s) * 2 + 1) * L + lane],
                                   ones)
            return 0
        lax.fori_loop(0, nvec, hbody, 0)

        def sbody(b, run):
            h = hist[pl.ds(b * L, L)]
            inc = plsc.cumsum(h)
            curs[pl.ds(b * L, L)] = run + (inc - h)
            tot = jnp.sum(h)
            return run + (((tot + CH - 1) >> SHIFT) << SHIFT)
        run_tot = lax.fori_loop(0, NBP, sbody, 0)

        def ebody(v, _):
            idx = (lane + v * L) * L
            cumb[pl.ds(v * L, L)] = plsc.load_gather(curs, [idx])
            return 0
        lax.fori_loop(0, NBP // L, ebody, 0)

        DUMT = SB * N
        def fbody(v, _):
            pos = v * L
            dums = DUMT + ((pos & 1023) + lane)
            r = pos & 15
            pada = ((DUMW + r) << 14) | (DUMW + r)
            stgi[pl.ds(pos, L)] = dums
            stga[pl.ds(pos, L)] = plsc.bitcast(
                jnp.full((L,), pada, jnp.int32), jnp.float32)
            return 0
        lax.fori_loop(0, (run_tot + L - 1) >> 4, fbody, 0)

        def pbody(i, _):
            s = srcb[pl.ds(i * L, L)]
            d = dstb[pl.ds(i * L, L)]
            c = cb[pl.ds(i * L, L)]
            bd = _blk(d)
            bs = _blk(s)
            ixw = ((bd * 2) * L) + lane
            slot = plsc.load_gather(curs, [ixw])
            plsc.store_scatter(curs, [ixw], slot + 1)
            plsc.store_scatter(stgi, [slot], (d - bd * SB) * N + s)
            plsc.store_scatter(stga, [slot], c)
            ixr = ((bs * 2 + 1) * L) + lane
            slot2 = plsc.load_gather(curs, [ixr])
            plsc.store_scatter(curs, [ixr], slot2 + 1)
            plsc.store_scatter(stgi, [slot2], (s - bs * SB) * N + d)
            plsc.store_scatter(stga, [slot2],
                               plsc.bitcast((s << 14) | d, jnp.float32))
            return 0
        lax.fori_loop(0, nvec, pbody, 0)

        pltpu.sync_copy(stgi, enti_hbm.at[pl.ds(wid * PWP, PWP)])
        pltpu.sync_copy(stga, enta_hbm.at[pl.ds(wid * PWP, PWP)])
        pltpu.sync_copy(cumb, cum_hbm.at[pl.ds(wid * NBP, NBP)])

    return k3


def _make_k4(N, NB, NBP, PWP, TSIZE, ZSPAN):
    NP16 = N + L
    ROWS = PWP // CH

    @functools.partial(
        pl.kernel, mesh=_mesh(),
        compiler_params=pltpu.CompilerParams(needs_layout_passes=False),
        out_type=jax.ShapeDtypeStruct((NW * N,), jnp.float32),
        scratch_types=[pltpu.VMEM((NW * NBP,), jnp.int32),
                       pltpu.VMEM((8, 1, CH), jnp.int32),
                       pltpu.VMEM((8, 1, CH), jnp.float32),
                       pltpu.VMEM((16, 1, CH), jnp.int32),
                       pltpu.VMEM((CH,), jnp.float32),
                       pltpu.VMEM((CH,), jnp.float32),
                       pltpu.VMEM((NP16,), jnp.float32),
                       pltpu.VMEM((2048,), jnp.float32),
                       pltpu.VMEM_SHARED((TSIZE,), jnp.float32),
                       pltpu.SemaphoreType.DMA,
                       pltpu.SemaphoreType.DMA,
                       pltpu.SemaphoreType.DMA,
                       pltpu.SemaphoreType.DMA,
                       pltpu.SemaphoreType.DMA,
                       pltpu.SemaphoreType.DMA,
                       pltpu.SemaphoreType.DMA,
                       pltpu.SemaphoreType.DMA],
    )
    def k4(enti_hbm, enta_hbm, cum_hbm, wpart_hbm,
           cums, ibuf, abuf, stash, g128, z128, wloc, zbuf, tmat,
           si0, si1, si2, si3, sa0, sa1, sa2, sa3):
        semi = [si0, si1, si2, si3]
        sema = [sa0, sa1, sa2, sa3]
        c = lax.axis_index("c")
        s = lax.axis_index("s")
        wid = s * NC + c
        pltpu.sync_copy(cum_hbm, cums)
        _zero_ref(zbuf, 2048 // L)
        _zero_ref(z128, CH // L)
        _zero_ref(wloc, NP16 // L)

        nz = ZSPAN // 2048
        zt = ZSPAN - nz * 2048

        def zb(i, _):
            pltpu.sync_copy(zbuf, tmat.at[pl.ds(s * ZSPAN + i * 2048, 2048)])
            return 0
        lax.fori_loop(0, nz, zb, 0)
        if zt:
            pltpu.sync_copy(zbuf.at[pl.ds(0, zt)],
                            tmat.at[pl.ds(s * ZSPAN + nz * 2048, zt)])
        plsc.subcore_barrier()

        def extract(k):
            v = plsc.load_gather(cums, [jnp.full((L,), k, jnp.int32)])
            return jnp.max(v)

        def ranges(bent):
            w0 = 2 * s
            w1 = 2 * s + 1
            lo0 = extract(w0 * NBP + bent)
            hi0 = extract(w0 * NBP + bent + 1)
            lo1 = extract(w1 * NBP + bent)
            hi1 = extract(w1 * NBP + bent + 1)
            return (w0 * ROWS + (lo0 >> SHIFT), (hi0 - lo0) >> SHIFT,
                    w1 * ROWS + (lo1 >> SHIFT), (hi1 - lo1) >> SHIFT)

        def hrow_of(rng):
            hr0, n0, hr1, n1 = rng

            def hrow(j):
                return jnp.where(j < n0, hr0 + j, hr1 + (j - n0))
            return hrow, n0 + n1

        def fire(rng, base, with_aux, s0=0):
            hrow, n = hrow_of(rng)
            for k in range(4):
                @pl.when(base + k < n)
                def _(k=k):
                    r = hrow(base + k)
                    pltpu.async_copy(enti_hbm.at[r], ibuf.at[s0 + k],
                                     semi[k])
                    if with_aux:
                        pltpu.async_copy(enta_hbm.at[r], abuf.at[s0 + k],
                                         sema[k])

        def merged(rng, with_aux, proc, prefired=False, s0=0):
            hrow, n = hrow_of(rng)

            def grp(g, _):
                base = g * 4
                pref = jnp.asarray(prefired, jnp.bool_)

                @pl.when(jnp.logical_or(g > 0, jnp.logical_not(pref)))
                def _():
                    fire(rng, base, with_aux, s0)
                for k in range(4):
                    @pl.when(base + k < n)
                    def _(k=k):
                        pltpu.make_async_copy(enti_hbm.at[0], ibuf.at[s0 + k],
                                              semi[k]).wait()
                        if with_aux:
                            pltpu.make_async_copy(enta_hbm.at[0],
                                                  abuf.at[s0 + k],
                                                  sema[k]).wait()
                        proc(s0 + k, base + k)
                return 0
            lax.fori_loop(0, (n + 3) >> 2, grp, 0)

        def proc_a(k, j):
            pltpu.sync_copy(abuf.at[k, 0], tmat.at[ibuf.at[k, 0]], add=True)

            @pl.when(j < 16)
            def _():
                for j2 in range(CH // L):
                    stash[j, 0, pl.ds(j2 * L, L)] = ibuf[k, 0,
                                                        pl.ds(j2 * L, L)]

        def proc_b(k, j):
            pltpu.sync_copy(tmat.at[ibuf.at[k, 0]], g128)
            for j2 in range(CH // L):
                co = g128[pl.ds(j2 * L, L)]
                ai = plsc.bitcast(abuf[k, 0, pl.ds(j2 * L, L)], jnp.int32)
                plsc.addupdate_scatter(wloc, [ai >> 14], co)
                plsc.addupdate_scatter(wloc, [ai & 0x3FFF], co)

        def phase_c(rng):
            hrow, n = hrow_of(rng)
            ns = jnp.minimum(n, 16)

            def cs(j, _):
                pltpu.sync_copy(z128, tmat.at[stash.at[j, 0]])
                return 0
            lax.fori_loop(0, ns, cs, 0)

            def ct(j, _):
                pltpu.sync_copy(enti_hbm.at[hrow(j)], ibuf.at[0])
                pltpu.sync_copy(z128, tmat.at[ibuf.at[0, 0]])
                return 0
            lax.fori_loop(ns, n, ct, 0)

        nq = (NB + 1) // 2 - c * (1 if NB % 2 else 0)

        def qloop(i, _):
            q = 2 * i + c
            r_w = ranges(2 * q)
            r_r = ranges(2 * q + 1)
            merged(r_w, True, proc_a, prefired=i > 0)
            fire(r_r, 0, True, s0=4)
            plsc.subcore_barrier()
            merged(r_r, True, proc_b, prefired=True, s0=4)
            plsc.subcore_barrier()
            phase_c(r_w)

            @pl.when(i + 1 < nq)
            def _():
                fire(ranges(2 * (2 * (i + 1) + c)), 0, True)
            plsc.subcore_barrier()
            return 0
        lax.fori_loop(0, nq, qloop, 0)
        pltpu.sync_copy(wloc.at[pl.ds(0, N)], wpart_hbm.at[pl.ds(wid * N, N)])

    return k4


def kernel(node_features, edge_list, kernel, attention_kernel, bias):
    N, F = node_features.shape
    D = kernel.shape[1]
    E = edge_list.shape[0]

    EW = E // NW
    assert EW * NW == E and EW % L == 0
    nfull, tail = divmod(EW, CH)
    NB = (N + SB - 1) // SB
    NBP = ((2 * NB + L - 1) // L) * L
    PWP = 2 * EW + NBP * (CH - 1)
    PWP = ((PWP + CH - 1) // CH) * CH
    TSIZE = SB * N + 1024
    ZSPAN = TSIZE // NS
    assert ZSPAN * NS == TSIZE and ZSPAN % 8 == 0
    NPAD = ((N + 511) // 512) * 512

    src = edge_list[:, 0]
    dst = edge_list[:, 1]
    ak2 = jnp.pad(attention_kernel.reshape(2, D).T, ((0, 0), (0, D - 2)))
    xpad = jnp.pad(node_features, ((0, NPAD - N), (0, 0)))

    keys, s12 = pl.pallas_call(
        _k0_body,
        grid=(NPAD // 512,),
        in_specs=[pl.BlockSpec((512, F), lambda i: (i, 0)),
                  pl.BlockSpec((F, D), lambda i: (0, 0)),
                  pl.BlockSpec((D, D), lambda i: (0, 0))],
        out_specs=[pl.BlockSpec((512, D), lambda i: (i, 0)),
                   pl.BlockSpec((512, D), lambda i: (i, 0))],
        out_shape=[jax.ShapeDtypeStruct((NPAD, D), jnp.float32),
                   jax.ShapeDtypeStruct((NPAD, D), jnp.float32)],
    )(xpad, kernel, ak2)

    s1 = s12[:N, 0]
    s2 = s12[:N, 1]

    parts = _make_k1(N, E, EW, nfull, tail)(src, dst, s1, s2)
    enti, enta, cum = _make_k3(N, E, EW, nfull, tail, NBP, PWP, N)(
        src, dst, s1, s2, parts)
    ROWS = PWP // CH
    wparts = _make_k4(N, NB, NBP, PWP, TSIZE, ZSPAN)(
        enti.reshape(NW * ROWS, 1, CH), enta.reshape(NW * ROWS, 1, CH), cum)

    wpad = jnp.pad(wparts.reshape(NW, N), ((0, 0), (0, NPAD - N)))
    out = pl.pallas_call(
        _k5_body,
        grid=(NPAD // 512,),
        in_specs=[pl.BlockSpec((512, D), lambda i: (i, 0)),
                  pl.BlockSpec((NW, 512), lambda i: (0, i)),
                  pl.BlockSpec((1, D), lambda i: (0, 0))],
        out_specs=pl.BlockSpec((512, D), lambda i: (i, 0)),
        out_shape=jax.ShapeDtypeStruct((NPAD, D), jnp.float32),
    )(keys, wpad, bias.reshape(1, D))
    return out[:N]
